# Initial kernel scaffold; baseline (speedup 1.0000x reference)
import jax
import jax.numpy as jnp
from jax.experimental import pallas as pl


def kernel(n_feats, e_feats, edge_index, node_graph_ids, params):
    raise NotImplementedError("write your pallas kernel here")


# speedup vs baseline: 1.1122x; 1.1122x over previous
import jax
import jax.numpy as jnp
from jax.experimental import pallas as pl
from jax.experimental.pallas import tpu as pltpu

N = 10000
E = 320000
G = 64
NF = 128
EF = 16
GF = 128


def _leaky(x):
    return jnp.where(x >= 0, x, 0.01 * x)


def _gru(x, h, p):
    gi = x @ p['W_ih'].T + p['b_ih']
    gh = h @ p['W_hh'].T + p['b_hh']
    i_r, i_z, i_n = jnp.split(gi, 3, axis=1)
    h_r, h_z, h_n = jnp.split(gh, 3, axis=1)
    r = jax.nn.sigmoid(i_r + h_r)
    z = jax.nn.sigmoid(i_z + h_z)
    n = jnp.tanh(i_n + r * h_n)
    return (1.0 - z) * n + z * h


def _segment_softmax(logits, seg, num_segments):
    m = jax.ops.segment_max(logits, seg, num_segments=num_segments)
    m = jnp.where(jnp.isfinite(m), m, 0.0)
    e = jnp.exp(logits - m[seg])
    s = jax.ops.segment_sum(e, seg, num_segments=num_segments)
    return e / (s[seg] + 1e-12)


def _hv_new_kernel(n_ref, w_ref, b_ref, o_ref):
    o_ref[...] = _leaky(n_ref[...] @ w_ref[...].T + b_ref[...])


def kernel(n_feats, e_feats, edge_index, node_graph_ids, params):
    src = edge_index[0]
    dst = edge_index[1]
    gc = params['gc']

    hv_new = pl.pallas_call(
        _hv_new_kernel,
        out_shape=jax.ShapeDtypeStruct((N, GF), jnp.float32),
        grid=(10,),
        in_specs=[
            pl.BlockSpec((N // 10, NF), lambda i: (i, 0)),
            pl.BlockSpec((GF, NF), lambda i: (0, 0)),
            pl.BlockSpec((GF,), lambda i: (0,)),
        ],
        out_specs=pl.BlockSpec((N // 10, GF), lambda i: (i, 0)),
    )(n_feats, gc['W_pn'], gc['b_pn'])

    he1 = _leaky(jnp.concatenate([n_feats[src], e_feats], axis=1) @ gc['W_pe1'].T + gc['b_pe1'])
    logits = _leaky(jnp.concatenate([hv_new[dst], he1], axis=1) @ gc['W_pe2'].T + gc['b_pe2'])
    a = _segment_softmax(logits, dst, N)
    msg = a * (he1 @ gc['W_et'].T + gc['b_et'])
    ctx = jax.nn.elu(jax.ops.segment_sum(msg, dst, num_segments=N))
    node_feats = jax.nn.relu(_gru(ctx, hv_new, gc))

    for lp in params['layers']:
        logits = _leaky(jnp.concatenate([node_feats[dst], node_feats[src]], axis=1) @ lp['W_pe'].T + lp['b_pe'])
        a = _segment_softmax(logits, dst, N)
        hv_proj = node_feats @ lp['W_pn'].T + lp['b_pn']
        ctx = jax.nn.elu(jax.ops.segment_sum(a * hv_proj[src], dst, num_segments=N))
        node_feats = jax.nn.relu(_gru(ctx, node_feats, lp))

    g_feats = jax.ops.segment_sum(node_feats, node_graph_ids, num_segments=G)
    for rp in params['readouts']:
        z = _leaky(jnp.concatenate([jax.nn.relu(g_feats)[node_graph_ids], node_feats], axis=1) @ rp['W_cl'].T + rp['b_cl'])
        a = _segment_softmax(z, node_graph_ids, G)
        hv = node_feats @ rp['W_pj'].T + rp['b_pj']
        g_repr = jax.nn.elu(jax.ops.segment_sum(a * hv, node_graph_ids, num_segments=G))
        g_feats = _gru(g_repr, g_feats, rp)
    return g_feats


# speedup vs baseline: 12.8956x; 11.5951x over previous
import functools

import jax
import jax.numpy as jnp
from jax import lax
from jax.experimental import pallas as pl
from jax.experimental.pallas import tpu as pltpu
from jax.experimental.pallas import tpu_sc as plsc

N = 10000
NP = 10240
E = 320000
G = 64
NF = 128
EF = 16
D = 128
NC = 2
NS = 16
NW = NC * NS
EB = 512
NBLK = E // EB
ER = E // 128
NSTRIPE = NP // NS
EBM = 2560
NBN = 1024


def _leaky(x):
    return jnp.maximum(x, 0.01 * x)


def _elu(x):
    return jnp.where(x > 0, x, jnp.exp(x) - 1.0)


def _tc_node_prep(x_ref, wpnT, bpn, wnodeT, wa, b2, hv_ref, p_ref, sd_ref):
    x = x_ref[...]
    hv = _leaky(x @ wpnT[...] + bpn[...])
    hv_ref[...] = hv
    p_ref[...] = x @ wnodeT[...]
    sd_ref[...] = hv @ wa[...] + b2[...]


def _tc_re1(ef_ref, weT, bpe1, o_ref):
    o_ref[...] = ef_ref[...] @ weT[...] + bpe1[...]


def _tc_edge_mm(psrc_ref, re1_ref, wetT, bet, wbr, m_ref, l_ref):
    he = _leaky(psrc_ref[...] + re1_ref[...])
    m_ref[...] = he @ wetT[...] + bet[...]
    l_ref[0, 0, :] = jnp.sum(he * wbr[...], axis=1)


def _gru_block(x, h, wihT, whhT, bih, bhh):
    gi = x @ wihT + bih
    gh = h @ whhT + bhh
    r = jax.nn.sigmoid(gi[:, :D] + gh[:, :D])
    z = jax.nn.sigmoid(gi[:, D:2 * D] + gh[:, D:2 * D])
    n = jnp.tanh(gi[:, 2 * D:] + r * gh[:, 2 * D:])
    return (1.0 - z) * n + z * h


def _tc_gru_prep(ctx_ref, h_ref, wihT, whhT, bih, bhh, wa, wb, bpe, wpnT, bpn,
                 nf_ref, sa_ref, sb_ref, hvp_ref):
    x = _elu(ctx_ref[0] + ctx_ref[1])
    nf = jnp.maximum(_gru_block(x, h_ref[...], wihT[...], whhT[...],
                                bih[...], bhh[...]), 0.0)
    nf_ref[...] = nf
    sa_ref[...] = nf @ wa[...] + bpe[...]
    sb_ref[...] = nf @ wb[...]
    hvp_ref[...] = nf @ wpnT[...] + bpn[...]


def _tc_gru_final(ctx_ref, h_ref, wihT, whhT, bih, bhh, nf_ref):
    x = _elu(ctx_ref[0] + ctx_ref[1])
    nf_ref[...] = jnp.maximum(
        _gru_block(x, h_ref[...], wihT[...], whhT[...], bih[...], bhh[...]),
        0.0)


def _tc_readout(nf_ref, gid_ref, *refs):
    (wpj0, bpj0, wacl0, wbcl0, bcl0, wih0, whh0, bih0, bhh0,
     wpj1, bpj1, wacl1, wbcl1, bcl1, wih1, whh1, bih1, bhh1, o_ref) = refs
    x = nf_ref[...]
    onehot = (gid_ref[...] == lax.broadcasted_iota(jnp.int32, (NP, G), 1)
              ).astype(jnp.float32)
    cdims = (((0,), (0,)), ((), ()))
    g = lax.dot_general(onehot, x, cdims)
    for (wpj, bpj, wacl, wbcl, bcl, wih, whh, bih, bhh) in (
            (wpj0, bpj0, wacl0, wbcl0, bcl0, wih0, whh0, bih0, bhh0),
            (wpj1, bpj1, wacl1, wbcl1, bcl1, wih1, whh1, bih1, bhh1)):
        ga = jnp.maximum(g, 0.0) @ wacl[...] + bcl[...]
        nb = x @ wbcl[...]
        z = _leaky(onehot @ ga + nb)
        e = jnp.exp(z)
        ssum = lax.dot_general(onehot, e, cdims)
        aa = e / (onehot @ ssum + 1e-12)
        hv = x @ wpj[...] + bpj[...]
        grp = _elu(lax.dot_general(onehot, aa * hv, cdims))
        g = _gru_block(grp, g, wih[...], whh[...], bih[...], bhh[...])
    o_ref[...] = g


def _sc_mesh():
    return plsc.VectorSubcoreMesh(
        core_axis_name="c", subcore_axis_name="s",
        num_cores=NC, num_subcores=NS)


_Z16F = functools.partial(jnp.zeros, (16,), jnp.float32)
_Z16I = functools.partial(jnp.zeros, (16,), jnp.int32)


def _nblocks(s):
    return jnp.where(s < 1, NBLK // NS + 1, NBLK // NS)


def _sc_gather_body(p_hbm, src3d, out_hbm, idxb, rows, sem):
    c = lax.axis_index("c")
    s = lax.axis_index("s")
    w = s * NC + c
    nblk = jnp.where(w < NBLK - (NBLK // NW) * NW, NBLK // NW + 1, NBLK // NW)

    @pl.loop(0, nblk)
    def _blk(i):
        b = w + i * NW
        pltpu.sync_copy(src3d.at[b], idxb)
        descs = [
            pltpu.async_copy(p_hbm.at[idxb.at[j]],
                             rows.at[pl.ds(j * 128, 128)], sem)
            for j in range(4)
        ]
        for d_ in descs:
            d_.wait()
        pltpu.sync_copy(rows, out_hbm.at[pl.ds(b * EB, EB)])


def _sc_zero_shared(s, rows, zs, s_sh, ctx_sh):
    z16 = _Z16F()

    @pl.loop(0, 8)
    def _z1(i):
        zs[pl.ds(i * 16, 16)] = z16

    @pl.loop(0, 128)
    def _z2(i):
        for q in range(D // 16):
            rows[i, pl.ds(q * 16, 16)] = z16

    for k in range(5):
        pltpu.sync_copy(zs, s_sh.at[pl.ds(s * 640 + k * 128, 128)])
        pltpu.sync_copy(rows, ctx_sh.at[pl.ds(s * 640 + k * 128, 128)])


def _sc_scale_rows(rows, sc):
    @pl.loop(0, 8)
    def _mul(g):
        a16 = sc[pl.ds(g * 16, 16)]
        for t in range(16):
            av = jnp.full((16,), a16[t], jnp.float32)
            j = g * 16 + t
            for q in range(D // 16):
                rows[j, pl.ds(q * 16, 16)] = rows[j, pl.ds(q * 16, 16)] * av


def _nblocks2(c, s):
    return jnp.where(c < 1, 20, jnp.where(s < 1, 20, 19))


def _sc_writeback(c, s, rows, elch, s_sh, ctx_sh, out_hbm):
    for k in range(5):
        r0 = s * 640 + k * 128
        pltpu.sync_copy(ctx_sh.at[pl.ds(r0, 128)], rows)
        pltpu.sync_copy(s_sh.at[pl.ds(r0, 128)], elch)

        @pl.loop(0, 8)
        def _inv(g):
            v = elch[pl.ds(g * 16, 16)]
            elch[pl.ds(g * 16, 16)] = 1.0 / (v + 1e-12)

        _sc_scale_rows(rows, elch)
        pltpu.sync_copy(rows, out_hbm.at[c, pl.ds(r0, 128)])


def _sc_ctx_getcontext(dst3d, sd_hbm, lcol3d, m_hbm, out_hbm,
                       idxd, lbuf, tabA, rows, elch, zs,
                       s_sh, ctx_sh, sem):
    c = lax.axis_index("c")
    s = lax.axis_index("s")
    pltpu.sync_copy(sd_hbm, tabA)
    _sc_zero_shared(s, rows, zs, s_sh, ctx_sh)
    plsc.subcore_barrier()

    @pl.loop(0, _nblocks(s))
    def _stage1(i):
        b = s + i * NS
        pltpu.sync_copy(dst3d.at[b], idxd)
        pltpu.sync_copy(lcol3d.at[b], lbuf)
        for j in range(4):
            for k in range(8):
                d16 = idxd[j, pl.ds(k * 16, 16)]
                x = plsc.load_gather(tabA, [d16]) + lbuf[j, pl.ds(k * 16, 16)]
                elch[pl.ds(k * 16, 16)] = jnp.exp(jnp.maximum(x, 0.01 * x))
            pltpu.sync_copy(elch, s_sh.at[idxd.at[j]], add=True)

    plsc.subcore_barrier()

    @pl.loop(0, _nblocks2(c, s))
    def _stage2(ii):
        i = c + 2 * ii
        b = s + i * NS
        pltpu.sync_copy(dst3d.at[b], idxd)
        pltpu.sync_copy(lcol3d.at[b], lbuf)
        for j in range(4):
            pltpu.sync_copy(m_hbm.at[pl.ds(b * EB + j * 128, 128)], rows)
            for k in range(8):
                d16 = idxd[j, pl.ds(k * 16, 16)]
                x = plsc.load_gather(tabA, [d16]) + lbuf[j, pl.ds(k * 16, 16)]
                elch[pl.ds(k * 16, 16)] = jnp.exp(jnp.maximum(x, 0.01 * x))
            _sc_scale_rows(rows, elch)
            pltpu.sync_copy(rows, ctx_sh.at[idxd.at[j]], add=True)

    plsc.subcore_barrier()
    _sc_writeback(c, s, rows, elch, s_sh, ctx_sh, out_hbm)


def _sc_ctx_layer(dst3d, src3d, sa_hbm, sb_hbm, hvp_hbm, out_hbm,
                  idxd, idxs, tabA, tabB, rows, elch, zs,
                  s_sh, ctx_sh, sem):
    c = lax.axis_index("c")
    s = lax.axis_index("s")
    pltpu.sync_copy(sa_hbm, tabA)
    pltpu.sync_copy(sb_hbm, tabB)
    _sc_zero_shared(s, rows, zs, s_sh, ctx_sh)
    plsc.subcore_barrier()

    @pl.loop(0, _nblocks(s))
    def _stage1(i):
        b = s + i * NS
        pltpu.sync_copy(dst3d.at[b], idxd)
        pltpu.sync_copy(src3d.at[b], idxs)
        for j in range(4):
            for k in range(8):
                d16 = idxd[j, pl.ds(k * 16, 16)]
                s16 = idxs[j, pl.ds(k * 16, 16)]
                x = (plsc.load_gather(tabA, [d16]) +
                     plsc.load_gather(tabB, [s16]))
                elch[pl.ds(k * 16, 16)] = jnp.exp(jnp.maximum(x, 0.01 * x))
            pltpu.sync_copy(elch, s_sh.at[idxd.at[j]], add=True)

    plsc.subcore_barrier()

    @pl.loop(0, _nblocks2(c, s))
    def _stage2(ii):
        i = c + 2 * ii
        b = s + i * NS
        pltpu.sync_copy(dst3d.at[b], idxd)
        pltpu.sync_copy(src3d.at[b], idxs)
        for j in range(4):
            pltpu.async_copy(hvp_hbm.at[idxs.at[j]], rows, sem).wait()
            for k in range(8):
                d16 = idxd[j, pl.ds(k * 16, 16)]
                s16 = idxs[j, pl.ds(k * 16, 16)]
                x = (plsc.load_gather(tabA, [d16]) +
                     plsc.load_gather(tabB, [s16]))
                elch[pl.ds(k * 16, 16)] = jnp.exp(jnp.maximum(x, 0.01 * x))
            _sc_scale_rows(rows, elch)
            pltpu.sync_copy(rows, ctx_sh.at[idxd.at[j]], add=True)

    plsc.subcore_barrier()
    _sc_writeback(c, s, rows, elch, s_sh, ctx_sh, out_hbm)


_CTX_SCRATCH_COMMON = [
    pltpu.VMEM((128, D), jnp.float32),
    pltpu.VMEM((128,), jnp.float32),
    pltpu.VMEM((128,), jnp.float32),
    pltpu.VMEM_SHARED((NP,), jnp.float32),
    pltpu.VMEM_SHARED((NP, D), jnp.float32),
    pltpu.SemaphoreType.DMA,
]


def _run_sc_gather(p, src3d):
    return pl.kernel(
        _sc_gather_body,
        out_type=jax.ShapeDtypeStruct((E, D), jnp.float32),
        mesh=_sc_mesh(),
        compiler_params=pltpu.CompilerParams(needs_layout_passes=False),
        scratch_types=[
            pltpu.VMEM((4, 128), jnp.int32),
            pltpu.VMEM((EB, D), jnp.float32),
            pltpu.SemaphoreType.DMA,
        ],
    )(p, src3d)


def _run_sc_getcontext(dst3d, sd, lcol3d, m):
    return pl.kernel(
        _sc_ctx_getcontext,
        out_type=jax.ShapeDtypeStruct((NC, NP, D), jnp.float32),
        mesh=_sc_mesh(),
        compiler_params=pltpu.CompilerParams(needs_layout_passes=False),
        scratch_types=[
            pltpu.VMEM((4, 128), jnp.int32),
            pltpu.VMEM((4, 128), jnp.float32),
            pltpu.VMEM((NP,), jnp.float32),
        ] + _CTX_SCRATCH_COMMON,
    )(dst3d, sd, lcol3d, m)


def _run_sc_layer(dst3d, src3d, sa, sb, hvp):
    return pl.kernel(
        _sc_ctx_layer,
        out_type=jax.ShapeDtypeStruct((NC, NP, D), jnp.float32),
        mesh=_sc_mesh(),
        compiler_params=pltpu.CompilerParams(needs_layout_passes=False),
        scratch_types=[
            pltpu.VMEM((4, 128), jnp.int32),
            pltpu.VMEM((4, 128), jnp.int32),
            pltpu.VMEM((NP,), jnp.float32),
            pltpu.VMEM((NP,), jnp.float32),
        ] + _CTX_SCRATCH_COMMON,
    )(dst3d, src3d, sa, sb, hvp)


def _full(shape):
    return pl.BlockSpec(shape, lambda *_: tuple(0 for _ in shape))


def kernel(n_feats, e_feats, edge_index, node_graph_ids, params):
    gc = params['gc']
    ei = edge_index.astype(jnp.int32)
    src3d = ei[0].reshape(NBLK, 4, 128)
    dst3d = ei[1].reshape(NBLK, 4, 128)
    nf_p = jnp.pad(n_feats, ((0, NP - N), (0, 0)))
    gid_p = jnp.pad(node_graph_ids.astype(jnp.int32), (0, NP - N),
                    constant_values=G).reshape(NP, 1)

    wpnT = gc['W_pn'].T
    bpn = gc['b_pn'].reshape(1, D)
    wnodeT = gc['W_pe1'][:, :NF].T
    weT = gc['W_pe1'][:, NF:].T
    bpe1 = gc['b_pe1'].reshape(1, D)
    wa0 = gc['W_pe2'][0, :D].reshape(D, 1)
    wbr0 = gc['W_pe2'][0, D:].reshape(1, D)
    b20 = gc['b_pe2'].reshape(1, 1)
    wetT = gc['W_et'].T
    bet = gc['b_et'].reshape(1, D)

    hv_new, p_tab, sd = pl.pallas_call(
        _tc_node_prep,
        grid=(NP // NBN,),
        in_specs=[
            pl.BlockSpec((NBN, D), lambda i: (i, 0)),
            _full((D, D)), _full((1, D)), _full((D, D)), _full((D, 1)),
            _full((1, 1)),
        ],
        out_specs=[
            pl.BlockSpec((NBN, D), lambda i: (i, 0)),
            pl.BlockSpec((NBN, D), lambda i: (i, 0)),
            pl.BlockSpec((NBN, 1), lambda i: (i, 0)),
        ],
        out_shape=[
            jax.ShapeDtypeStruct((NP, D), jnp.float32),
            jax.ShapeDtypeStruct((NP, D), jnp.float32),
            jax.ShapeDtypeStruct((NP, 1), jnp.float32),
        ],
    )(nf_p, wpnT, bpn, wnodeT, wa0, b20)

    re1 = pl.pallas_call(
        _tc_re1,
        grid=(E // EBM,),
        in_specs=[
            pl.BlockSpec((EBM, EF), lambda i: (i, 0)),
            _full((EF, D)), _full((1, D)),
        ],
        out_specs=pl.BlockSpec((EBM, D), lambda i: (i, 0)),
        out_shape=jax.ShapeDtypeStruct((E, D), jnp.float32),
    )(e_feats, weT, bpe1)

    psrc = _run_sc_gather(p_tab, src3d)

    m_mat, lcol3 = pl.pallas_call(
        _tc_edge_mm,
        grid=(E // EBM,),
        in_specs=[
            pl.BlockSpec((EBM, D), lambda i: (i, 0)),
            pl.BlockSpec((EBM, D), lambda i: (i, 0)),
            _full((D, D)), _full((1, D)), _full((1, D)),
        ],
        out_specs=[
            pl.BlockSpec((EBM, D), lambda i: (i, 0)),
            pl.BlockSpec((1, 1, EBM), lambda i: (i, 0, 0)),
        ],
        out_shape=[
            jax.ShapeDtypeStruct((E, D), jnp.float32),
            jax.ShapeDtypeStruct((E // EBM, 1, EBM), jnp.float32),
        ],
    )(psrc, re1, wetT, bet, wbr0)
    lcol3d = lcol3.reshape(NBLK, 4, 128)

    ctx = _run_sc_getcontext(dst3d, sd.reshape(NP), lcol3d, m_mat)

    h = hv_new
    for li, lp in enumerate(params['layers']):
        wa = lp['W_pe'][0, :D].reshape(D, 1)
        wb = lp['W_pe'][0, D:].reshape(D, 1)
        bpe = lp['b_pe'].reshape(1, 1)
        gp = params['gc'] if li == 0 else params['layers'][li - 1]
        nf, sa, sb, hvp = pl.pallas_call(
            _tc_gru_prep,
            grid=(NP // NBN,),
            in_specs=[
                pl.BlockSpec((NC, NBN, D), lambda i: (0, i, 0)),
                pl.BlockSpec((NBN, D), lambda i: (i, 0)),
                _full((D, 3 * D)), _full((D, 3 * D)),
                _full((1, 3 * D)), _full((1, 3 * D)),
                _full((D, 1)), _full((D, 1)), _full((1, 1)),
                _full((D, D)), _full((1, D)),
            ],
            out_specs=[
                pl.BlockSpec((NBN, D), lambda i: (i, 0)),
                pl.BlockSpec((NBN, 1), lambda i: (i, 0)),
                pl.BlockSpec((NBN, 1), lambda i: (i, 0)),
                pl.BlockSpec((NBN, D), lambda i: (i, 0)),
            ],
            out_shape=[
                jax.ShapeDtypeStruct((NP, D), jnp.float32),
                jax.ShapeDtypeStruct((NP, 1), jnp.float32),
                jax.ShapeDtypeStruct((NP, 1), jnp.float32),
                jax.ShapeDtypeStruct((NP, D), jnp.float32),
            ],
        )(ctx, h, gp['W_ih'].T, gp['W_hh'].T,
          gp['b_ih'].reshape(1, 3 * D), gp['b_hh'].reshape(1, 3 * D),
          wa, wb, bpe, lp['W_pn'].T, lp['b_pn'].reshape(1, D))
        ctx = _run_sc_layer(dst3d, src3d, sa.reshape(NP), sb.reshape(NP), hvp)
        h = nf

    lp_last = params['layers'][-1]
    nf_final = pl.pallas_call(
        _tc_gru_final,
        grid=(NP // NBN,),
        in_specs=[
            pl.BlockSpec((NC, NBN, D), lambda i: (0, i, 0)),
            pl.BlockSpec((NBN, D), lambda i: (i, 0)),
            _full((D, 3 * D)), _full((D, 3 * D)),
            _full((1, 3 * D)), _full((1, 3 * D)),
        ],
        out_specs=pl.BlockSpec((NBN, D), lambda i: (i, 0)),
        out_shape=jax.ShapeDtypeStruct((NP, D), jnp.float32),
    )(ctx, h, lp_last['W_ih'].T, lp_last['W_hh'].T,
      lp_last['b_ih'].reshape(1, 3 * D), lp_last['b_hh'].reshape(1, 3 * D))

    ro_args = []
    for rp in params['readouts']:
        ro_args += [
            rp['W_pj'].T, rp['b_pj'].reshape(1, D),
            rp['W_cl'][0, :D].reshape(D, 1),
            rp['W_cl'][0, D:].reshape(D, 1),
            rp['b_cl'].reshape(1, 1),
            rp['W_ih'].T, rp['W_hh'].T,
            rp['b_ih'].reshape(1, 3 * D), rp['b_hh'].reshape(1, 3 * D),
        ]

    ro_specs = [
        _full((D, D)), _full((1, D)), _full((D, 1)), _full((D, 1)),
        _full((1, 1)),
        _full((D, 3 * D)), _full((D, 3 * D)), _full((1, 3 * D)),
        _full((1, 3 * D)),
    ]
    g_feats = pl.pallas_call(
        _tc_readout,
        in_specs=[_full((NP, D)), _full((NP, 1))] + ro_specs + ro_specs,
        out_specs=_full((G, D)),
        out_shape=jax.ShapeDtypeStruct((G, D), jnp.float32),
    )(nf_final, gid_p, *ro_args)
    return g_feats


# speedup vs baseline: 13.4193x; 1.0406x over previous
import functools

import jax
import jax.numpy as jnp
from jax import lax
from jax.experimental import pallas as pl
from jax.experimental.pallas import tpu as pltpu
from jax.experimental.pallas import tpu_sc as plsc

N = 10000
NP = 10240
E = 320000
G = 64
NF = 128
EF = 16
D = 128
NC = 2
NS = 16
NW = NC * NS
EB = 512
NBLK = E // EB
ER = E // 128
NSTRIPE = NP // NS
EBM = 2560
NBN = 1024


def _leaky(x):
    return jnp.maximum(x, 0.01 * x)


def _elu(x):
    return jnp.where(x > 0, x, jnp.exp(x) - 1.0)


def _tc_node_prep(x_ref, wpnT, bpn, wnodeT, wa, b2, hv_ref, p_ref, sd_ref):
    x = x_ref[...]
    hv = _leaky(x @ wpnT[...] + bpn[...])
    hv_ref[...] = hv
    p_ref[...] = x @ wnodeT[...]
    sd_ref[...] = hv @ wa[...] + b2[...]


def _tc_edge_mm(psrc_ref, ef_ref, weT, bpe1, wetT, bet, wbr, m_ref, l_ref):
    he = _leaky(psrc_ref[...] + ef_ref[...] @ weT[...] + bpe1[...])
    m_ref[...] = he @ wetT[...] + bet[...]
    l_ref[0, 0, :] = jnp.sum(he * wbr[...], axis=1)


def _gru_block(x, h, wihT, whhT, bih, bhh):
    gi = x @ wihT + bih
    gh = h @ whhT + bhh
    r = jax.nn.sigmoid(gi[:, :D] + gh[:, :D])
    z = jax.nn.sigmoid(gi[:, D:2 * D] + gh[:, D:2 * D])
    n = jnp.tanh(gi[:, 2 * D:] + r * gh[:, 2 * D:])
    return (1.0 - z) * n + z * h


def _tc_gru_prep(ctx_ref, h_ref, wihT, whhT, bih, bhh, wa, wb, bpe, wpnT, bpn,
                 nf_ref, sa_ref, sb_ref, hvp_ref):
    x = _elu(ctx_ref[0] + ctx_ref[1])
    nf = jnp.maximum(_gru_block(x, h_ref[...], wihT[...], whhT[...],
                                bih[...], bhh[...]), 0.0)
    nf_ref[...] = nf
    sa_ref[...] = nf @ wa[...] + bpe[...]
    sb_ref[...] = nf @ wb[...]
    hvp_ref[...] = nf @ wpnT[...] + bpn[...]


def _tc_gru_final(ctx_ref, h_ref, wihT, whhT, bih, bhh, nf_ref):
    x = _elu(ctx_ref[0] + ctx_ref[1])
    nf_ref[...] = jnp.maximum(
        _gru_block(x, h_ref[...], wihT[...], whhT[...], bih[...], bhh[...]),
        0.0)


def _tc_readout(nf_ref, gid_ref, *refs):
    (wpj0, bpj0, wacl0, wbcl0, bcl0, wih0, whh0, bih0, bhh0,
     wpj1, bpj1, wacl1, wbcl1, bcl1, wih1, whh1, bih1, bhh1, o_ref) = refs
    x = nf_ref[...]
    onehot = (gid_ref[...] == lax.broadcasted_iota(jnp.int32, (NP, G), 1)
              ).astype(jnp.float32)
    cdims = (((0,), (0,)), ((), ()))
    g = lax.dot_general(onehot, x, cdims)
    for (wpj, bpj, wacl, wbcl, bcl, wih, whh, bih, bhh) in (
            (wpj0, bpj0, wacl0, wbcl0, bcl0, wih0, whh0, bih0, bhh0),
            (wpj1, bpj1, wacl1, wbcl1, bcl1, wih1, whh1, bih1, bhh1)):
        ga = jnp.maximum(g, 0.0) @ wacl[...] + bcl[...]
        nb = x @ wbcl[...]
        z = _leaky(onehot @ ga + nb)
        e = jnp.exp(z)
        ssum = lax.dot_general(onehot, e, cdims)
        aa = e / (onehot @ ssum + 1e-12)
        hv = x @ wpj[...] + bpj[...]
        grp = _elu(lax.dot_general(onehot, aa * hv, cdims))
        g = _gru_block(grp, g, wih[...], whh[...], bih[...], bhh[...])
    o_ref[...] = g


def _sc_mesh():
    return plsc.VectorSubcoreMesh(
        core_axis_name="c", subcore_axis_name="s",
        num_cores=NC, num_subcores=NS)


_Z16F = functools.partial(jnp.zeros, (16,), jnp.float32)
_Z16I = functools.partial(jnp.zeros, (16,), jnp.int32)


def _nblocks(s):
    return jnp.where(s < 1, NBLK // NS + 1, NBLK // NS)


def _sc_gather_body(p_hbm, src3d, out_hbm, idxb, rows, sem):
    c = lax.axis_index("c")
    s = lax.axis_index("s")
    w = s * NC + c
    nblk = jnp.where(w < NBLK - (NBLK // NW) * NW, NBLK // NW + 1, NBLK // NW)

    @pl.loop(0, nblk)
    def _blk(i):
        b = w + i * NW
        pltpu.sync_copy(src3d.at[b], idxb)
        descs = [
            pltpu.async_copy(p_hbm.at[idxb.at[j]],
                             rows.at[pl.ds(j * 128, 128)], sem)
            for j in range(4)
        ]
        for d_ in descs:
            d_.wait()
        pltpu.sync_copy(rows, out_hbm.at[pl.ds(b * EB, EB)])


def _sc_zero_shared(s, rows, zs, s_sh, ctx_sh):
    z16 = _Z16F()

    @pl.loop(0, 8)
    def _z1(i):
        zs[pl.ds(i * 16, 16)] = z16

    @pl.loop(0, 128)
    def _z2(i):
        for q in range(D // 16):
            rows[i, pl.ds(q * 16, 16)] = z16

    for k in range(5):
        pltpu.sync_copy(zs, s_sh.at[pl.ds(s * 640 + k * 128, 128)])
        pltpu.sync_copy(rows, ctx_sh.at[pl.ds(s * 640 + k * 128, 128)])


def _sc_scale_rows(rows, sc):
    @pl.loop(0, 8)
    def _mul(g):
        a16 = sc[pl.ds(g * 16, 16)]
        for t in range(16):
            av = jnp.full((16,), a16[t], jnp.float32)
            j = g * 16 + t
            for q in range(D // 16):
                rows[j, pl.ds(q * 16, 16)] = rows[j, pl.ds(q * 16, 16)] * av


def _nblocks2(c, s):
    return jnp.where(c < 1, 20, jnp.where(s < 1, 20, 19))


def _sc_writeback(c, s, rows, elch, s_sh, ctx_sh, out_hbm):
    for k in range(5):
        r0 = s * 640 + k * 128
        pltpu.sync_copy(ctx_sh.at[pl.ds(r0, 128)], rows)
        pltpu.sync_copy(s_sh.at[pl.ds(r0, 128)], elch)

        @pl.loop(0, 8)
        def _inv(g):
            v = elch[pl.ds(g * 16, 16)]
            elch[pl.ds(g * 16, 16)] = 1.0 / (v + 1e-12)

        _sc_scale_rows(rows, elch)
        pltpu.sync_copy(rows, out_hbm.at[c, pl.ds(r0, 128)])


def _sc_ctx_getcontext(dst3d, sd_hbm, lcol3d, m_hbm, out_hbm,
                       idxd, lbuf, tabA, rows, elch, zs,
                       s_sh, ctx_sh, sem):
    c = lax.axis_index("c")
    s = lax.axis_index("s")
    pltpu.sync_copy(sd_hbm, tabA)
    _sc_zero_shared(s, rows, zs, s_sh, ctx_sh)
    plsc.subcore_barrier()

    @pl.loop(0, _nblocks(s))
    def _stage1(i):
        b = s + i * NS
        pltpu.sync_copy(dst3d.at[b], idxd)
        pltpu.sync_copy(lcol3d.at[b], lbuf)
        for j in range(4):
            for k in range(8):
                d16 = idxd[j, pl.ds(k * 16, 16)]
                x = plsc.load_gather(tabA, [d16]) + lbuf[j, pl.ds(k * 16, 16)]
                elch[pl.ds(k * 16, 16)] = jnp.exp(jnp.maximum(x, 0.01 * x))
            pltpu.sync_copy(elch, s_sh.at[idxd.at[j]], add=True)

    plsc.subcore_barrier()

    @pl.loop(0, _nblocks2(c, s))
    def _stage2(ii):
        i = c + 2 * ii
        b = s + i * NS
        pltpu.sync_copy(dst3d.at[b], idxd)
        pltpu.sync_copy(lcol3d.at[b], lbuf)
        for j in range(4):
            pltpu.sync_copy(m_hbm.at[pl.ds(b * EB + j * 128, 128)], rows)
            for k in range(8):
                d16 = idxd[j, pl.ds(k * 16, 16)]
                x = plsc.load_gather(tabA, [d16]) + lbuf[j, pl.ds(k * 16, 16)]
                elch[pl.ds(k * 16, 16)] = jnp.exp(jnp.maximum(x, 0.01 * x))
            _sc_scale_rows(rows, elch)
            pltpu.sync_copy(rows, ctx_sh.at[idxd.at[j]], add=True)

    plsc.subcore_barrier()
    _sc_writeback(c, s, rows, elch, s_sh, ctx_sh, out_hbm)


def _sc_ctx_layer(dst3d, src3d, sa_hbm, sb_hbm, hvp_hbm, out_hbm,
                  idxd, idxs, tabA, tabB, rows, elch, zs,
                  s_sh, ctx_sh, sem):
    c = lax.axis_index("c")
    s = lax.axis_index("s")
    pltpu.sync_copy(sa_hbm, tabA)
    pltpu.sync_copy(sb_hbm, tabB)
    _sc_zero_shared(s, rows, zs, s_sh, ctx_sh)
    plsc.subcore_barrier()

    @pl.loop(0, _nblocks(s))
    def _stage1(i):
        b = s + i * NS
        pltpu.sync_copy(dst3d.at[b], idxd)
        pltpu.sync_copy(src3d.at[b], idxs)
        for j in range(4):
            for k in range(8):
                d16 = idxd[j, pl.ds(k * 16, 16)]
                s16 = idxs[j, pl.ds(k * 16, 16)]
                x = (plsc.load_gather(tabA, [d16]) +
                     plsc.load_gather(tabB, [s16]))
                elch[pl.ds(k * 16, 16)] = jnp.exp(jnp.maximum(x, 0.01 * x))
            pltpu.sync_copy(elch, s_sh.at[idxd.at[j]], add=True)

    plsc.subcore_barrier()

    @pl.loop(0, _nblocks2(c, s))
    def _stage2(ii):
        i = c + 2 * ii
        b = s + i * NS
        pltpu.sync_copy(dst3d.at[b], idxd)
        pltpu.sync_copy(src3d.at[b], idxs)
        for j in range(4):
            pltpu.async_copy(hvp_hbm.at[idxs.at[j]], rows, sem).wait()
            for k in range(8):
                d16 = idxd[j, pl.ds(k * 16, 16)]
                s16 = idxs[j, pl.ds(k * 16, 16)]
                x = (plsc.load_gather(tabA, [d16]) +
                     plsc.load_gather(tabB, [s16]))
                elch[pl.ds(k * 16, 16)] = jnp.exp(jnp.maximum(x, 0.01 * x))
            _sc_scale_rows(rows, elch)
            pltpu.sync_copy(rows, ctx_sh.at[idxd.at[j]], add=True)

    plsc.subcore_barrier()
    _sc_writeback(c, s, rows, elch, s_sh, ctx_sh, out_hbm)


_CTX_SCRATCH_COMMON = [
    pltpu.VMEM((128, D), jnp.float32),
    pltpu.VMEM((128,), jnp.float32),
    pltpu.VMEM((128,), jnp.float32),
    pltpu.VMEM_SHARED((NP,), jnp.float32),
    pltpu.VMEM_SHARED((NP, D), jnp.float32),
    pltpu.SemaphoreType.DMA,
]


def _run_sc_gather(p, src3d):
    return pl.kernel(
        _sc_gather_body,
        out_type=jax.ShapeDtypeStruct((E, D), jnp.float32),
        mesh=_sc_mesh(),
        compiler_params=pltpu.CompilerParams(needs_layout_passes=False),
        scratch_types=[
            pltpu.VMEM((4, 128), jnp.int32),
            pltpu.VMEM((EB, D), jnp.float32),
            pltpu.SemaphoreType.DMA,
        ],
    )(p, src3d)


def _run_sc_getcontext(dst3d, sd, lcol3d, m):
    return pl.kernel(
        _sc_ctx_getcontext,
        out_type=jax.ShapeDtypeStruct((NC, NP, D), jnp.float32),
        mesh=_sc_mesh(),
        compiler_params=pltpu.CompilerParams(needs_layout_passes=False),
        scratch_types=[
            pltpu.VMEM((4, 128), jnp.int32),
            pltpu.VMEM((4, 128), jnp.float32),
            pltpu.VMEM((NP,), jnp.float32),
        ] + _CTX_SCRATCH_COMMON,
    )(dst3d, sd, lcol3d, m)


def _run_sc_layer(dst3d, src3d, sa, sb, hvp):
    return pl.kernel(
        _sc_ctx_layer,
        out_type=jax.ShapeDtypeStruct((NC, NP, D), jnp.float32),
        mesh=_sc_mesh(),
        compiler_params=pltpu.CompilerParams(needs_layout_passes=False),
        scratch_types=[
            pltpu.VMEM((4, 128), jnp.int32),
            pltpu.VMEM((4, 128), jnp.int32),
            pltpu.VMEM((NP,), jnp.float32),
            pltpu.VMEM((NP,), jnp.float32),
        ] + _CTX_SCRATCH_COMMON,
    )(dst3d, src3d, sa, sb, hvp)


def _full(shape):
    return pl.BlockSpec(shape, lambda *_: tuple(0 for _ in shape))


def kernel(n_feats, e_feats, edge_index, node_graph_ids, params):
    gc = params['gc']
    ei = edge_index.astype(jnp.int32)
    src3d = ei[0].reshape(NBLK, 4, 128)
    dst3d = ei[1].reshape(NBLK, 4, 128)
    nf_p = jnp.pad(n_feats, ((0, NP - N), (0, 0)))
    gid_p = jnp.pad(node_graph_ids.astype(jnp.int32), (0, NP - N),
                    constant_values=G).reshape(NP, 1)

    wpnT = gc['W_pn'].T
    bpn = gc['b_pn'].reshape(1, D)
    wnodeT = gc['W_pe1'][:, :NF].T
    weT = gc['W_pe1'][:, NF:].T
    bpe1 = gc['b_pe1'].reshape(1, D)
    wa0 = gc['W_pe2'][0, :D].reshape(D, 1)
    wbr0 = gc['W_pe2'][0, D:].reshape(1, D)
    b20 = gc['b_pe2'].reshape(1, 1)
    wetT = gc['W_et'].T
    bet = gc['b_et'].reshape(1, D)

    hv_new, p_tab, sd = pl.pallas_call(
        _tc_node_prep,
        grid=(NP // NBN,),
        in_specs=[
            pl.BlockSpec((NBN, D), lambda i: (i, 0)),
            _full((D, D)), _full((1, D)), _full((D, D)), _full((D, 1)),
            _full((1, 1)),
        ],
        out_specs=[
            pl.BlockSpec((NBN, D), lambda i: (i, 0)),
            pl.BlockSpec((NBN, D), lambda i: (i, 0)),
            pl.BlockSpec((NBN, 1), lambda i: (i, 0)),
        ],
        out_shape=[
            jax.ShapeDtypeStruct((NP, D), jnp.float32),
            jax.ShapeDtypeStruct((NP, D), jnp.float32),
            jax.ShapeDtypeStruct((NP, 1), jnp.float32),
        ],
    )(nf_p, wpnT, bpn, wnodeT, wa0, b20)

    psrc = _run_sc_gather(p_tab, src3d)

    m_mat, lcol3 = pl.pallas_call(
        _tc_edge_mm,
        grid=(E // EBM,),
        in_specs=[
            pl.BlockSpec((EBM, D), lambda i: (i, 0)),
            pl.BlockSpec((EBM, EF), lambda i: (i, 0)),
            _full((EF, D)), _full((1, D)),
            _full((D, D)), _full((1, D)), _full((1, D)),
        ],
        out_specs=[
            pl.BlockSpec((EBM, D), lambda i: (i, 0)),
            pl.BlockSpec((1, 1, EBM), lambda i: (i, 0, 0)),
        ],
        out_shape=[
            jax.ShapeDtypeStruct((E, D), jnp.float32),
            jax.ShapeDtypeStruct((E // EBM, 1, EBM), jnp.float32),
        ],
    )(psrc, e_feats, weT, bpe1, wetT, bet, wbr0)
    lcol3d = lcol3.reshape(NBLK, 4, 128)

    ctx = _run_sc_getcontext(dst3d, sd.reshape(NP), lcol3d, m_mat)

    h = hv_new
    for li, lp in enumerate(params['layers']):
        wa = lp['W_pe'][0, :D].reshape(D, 1)
        wb = lp['W_pe'][0, D:].reshape(D, 1)
        bpe = lp['b_pe'].reshape(1, 1)
        gp = params['gc'] if li == 0 else params['layers'][li - 1]
        nf, sa, sb, hvp = pl.pallas_call(
            _tc_gru_prep,
            grid=(NP // NBN,),
            in_specs=[
                pl.BlockSpec((NC, NBN, D), lambda i: (0, i, 0)),
                pl.BlockSpec((NBN, D), lambda i: (i, 0)),
                _full((D, 3 * D)), _full((D, 3 * D)),
                _full((1, 3 * D)), _full((1, 3 * D)),
                _full((D, 1)), _full((D, 1)), _full((1, 1)),
                _full((D, D)), _full((1, D)),
            ],
            out_specs=[
                pl.BlockSpec((NBN, D), lambda i: (i, 0)),
                pl.BlockSpec((NBN, 1), lambda i: (i, 0)),
                pl.BlockSpec((NBN, 1), lambda i: (i, 0)),
                pl.BlockSpec((NBN, D), lambda i: (i, 0)),
            ],
            out_shape=[
                jax.ShapeDtypeStruct((NP, D), jnp.float32),
                jax.ShapeDtypeStruct((NP, 1), jnp.float32),
                jax.ShapeDtypeStruct((NP, 1), jnp.float32),
                jax.ShapeDtypeStruct((NP, D), jnp.float32),
            ],
        )(ctx, h, gp['W_ih'].T, gp['W_hh'].T,
          gp['b_ih'].reshape(1, 3 * D), gp['b_hh'].reshape(1, 3 * D),
          wa, wb, bpe, lp['W_pn'].T, lp['b_pn'].reshape(1, D))
        ctx = _run_sc_layer(dst3d, src3d, sa.reshape(NP), sb.reshape(NP), hvp)
        h = nf

    lp_last = params['layers'][-1]
    nf_final = pl.pallas_call(
        _tc_gru_final,
        grid=(NP // NBN,),
        in_specs=[
            pl.BlockSpec((NC, NBN, D), lambda i: (0, i, 0)),
            pl.BlockSpec((NBN, D), lambda i: (i, 0)),
            _full((D, 3 * D)), _full((D, 3 * D)),
            _full((1, 3 * D)), _full((1, 3 * D)),
        ],
        out_specs=pl.BlockSpec((NBN, D), lambda i: (i, 0)),
        out_shape=jax.ShapeDtypeStruct((NP, D), jnp.float32),
    )(ctx, h, lp_last['W_ih'].T, lp_last['W_hh'].T,
      lp_last['b_ih'].reshape(1, 3 * D), lp_last['b_hh'].reshape(1, 3 * D))

    ro_args = []
    for rp in params['readouts']:
        ro_args += [
            rp['W_pj'].T, rp['b_pj'].reshape(1, D),
            rp['W_cl'][0, :D].reshape(D, 1),
            rp['W_cl'][0, D:].reshape(D, 1),
            rp['b_cl'].reshape(1, 1),
            rp['W_ih'].T, rp['W_hh'].T,
            rp['b_ih'].reshape(1, 3 * D), rp['b_hh'].reshape(1, 3 * D),
        ]

    ro_specs = [
        _full((D, D)), _full((1, D)), _full((D, 1)), _full((D, 1)),
        _full((1, 1)),
        _full((D, 3 * D)), _full((D, 3 * D)), _full((1, 3 * D)),
        _full((1, 3 * D)),
    ]
    g_feats = pl.pallas_call(
        _tc_readout,
        in_specs=[_full((NP, D)), _full((NP, 1))] + ro_specs + ro_specs,
        out_specs=_full((G, D)),
        out_shape=jax.ShapeDtypeStruct((G, D), jnp.float32),
    )(nf_final, gid_p, *ro_args)
    return g_feats


# speedup vs baseline: 15.6807x; 1.1685x over previous
import functools

import jax
import jax.numpy as jnp
from jax import lax
from jax.experimental import pallas as pl
from jax.experimental.pallas import tpu as pltpu
from jax.experimental.pallas import tpu_sc as plsc

N = 10000
NP = 10240
E = 320000
G = 64
NF = 128
EF = 16
D = 128
NC = 2
NS = 16
NW = NC * NS
EB = 512
NBLK = E // EB
ER = E // 128
NSTRIPE = NP // NS
EBM = 2560
NBN = 1024


def _leaky(x):
    return jnp.maximum(x, 0.01 * x)


def _elu(x):
    return jnp.where(x > 0, x, jnp.exp(x) - 1.0)


def _tc_node_prep(x_ref, wpnT, bpn, wnodeT, wa, b2, hv_ref, p_ref, sd_ref):
    x = x_ref[...]
    hv = _leaky(x @ wpnT[...] + bpn[...])
    hv_ref[...] = hv
    p_ref[...] = x @ wnodeT[...]
    sd_ref[...] = hv @ wa[...] + b2[...]


def _tc_edge_mm(psrc_ref, ef_ref, weT, bpe1, wetT, bet, wbr, m_ref, l_ref):
    he = _leaky(psrc_ref[...] + ef_ref[...] @ weT[...] + bpe1[...])
    m_ref[...] = he @ wetT[...] + bet[...]
    l_ref[0, 0, :] = jnp.sum(he * wbr[...], axis=1)


def _gru_block(x, h, wihT, whhT, bih, bhh):
    gi = x @ wihT + bih
    gh = h @ whhT + bhh
    r = jax.nn.sigmoid(gi[:, :D] + gh[:, :D])
    z = jax.nn.sigmoid(gi[:, D:2 * D] + gh[:, D:2 * D])
    n = jnp.tanh(gi[:, 2 * D:] + r * gh[:, 2 * D:])
    return (1.0 - z) * n + z * h


def _tc_gru_prep(ctx_ref, sden_ref, h_ref, wihT, whhT, bih, bhh, wa, wb, bpe,
                 wpnT, bpn, nf_ref, sa_ref, sb_ref, hvp_ref):
    x = _elu((ctx_ref[0] + ctx_ref[1]) /
             (sden_ref[0] + sden_ref[1] + 1e-12))
    nf = jnp.maximum(_gru_block(x, h_ref[...], wihT[...], whhT[...],
                                bih[...], bhh[...]), 0.0)
    nf_ref[...] = nf
    sa_ref[...] = nf @ wa[...] + bpe[...]
    sb_ref[...] = nf @ wb[...]
    hvp_ref[...] = nf @ wpnT[...] + bpn[...]


def _tc_gru_final(ctx_ref, sden_ref, h_ref, wihT, whhT, bih, bhh, nf_ref):
    x = _elu((ctx_ref[0] + ctx_ref[1]) /
             (sden_ref[0] + sden_ref[1] + 1e-12))
    nf_ref[...] = jnp.maximum(
        _gru_block(x, h_ref[...], wihT[...], whhT[...], bih[...], bhh[...]),
        0.0)


def _tc_readout(nf_ref, gid_ref, *refs):
    (wpj0, bpj0, wacl0, wbcl0, bcl0, wih0, whh0, bih0, bhh0,
     wpj1, bpj1, wacl1, wbcl1, bcl1, wih1, whh1, bih1, bhh1, o_ref) = refs
    x = nf_ref[...]
    onehot = (gid_ref[...] == lax.broadcasted_iota(jnp.int32, (NP, G), 1)
              ).astype(jnp.float32)
    cdims = (((0,), (0,)), ((), ()))
    g = lax.dot_general(onehot, x, cdims)
    for (wpj, bpj, wacl, wbcl, bcl, wih, whh, bih, bhh) in (
            (wpj0, bpj0, wacl0, wbcl0, bcl0, wih0, whh0, bih0, bhh0),
            (wpj1, bpj1, wacl1, wbcl1, bcl1, wih1, whh1, bih1, bhh1)):
        ga = jnp.maximum(g, 0.0) @ wacl[...] + bcl[...]
        nb = x @ wbcl[...]
        z = _leaky(onehot @ ga + nb)
        e = jnp.exp(z)
        ssum = lax.dot_general(onehot, e, cdims)
        aa = e / (onehot @ ssum + 1e-12)
        hv = x @ wpj[...] + bpj[...]
        grp = _elu(lax.dot_general(onehot, aa * hv, cdims))
        g = _gru_block(grp, g, wih[...], whh[...], bih[...], bhh[...])
    o_ref[...] = g


def _sc_mesh():
    return plsc.VectorSubcoreMesh(
        core_axis_name="c", subcore_axis_name="s",
        num_cores=NC, num_subcores=NS)


_Z16F = functools.partial(jnp.zeros, (16,), jnp.float32)
_Z16I = functools.partial(jnp.zeros, (16,), jnp.int32)


def _nblocks(s):
    return jnp.where(s < 1, NBLK // NS + 1, NBLK // NS)


def _sc_gather_body(p_hbm, src3d, out_hbm, idxb, rows, sem):
    c = lax.axis_index("c")
    s = lax.axis_index("s")
    w = s * NC + c
    nblk = jnp.where(w < NBLK - (NBLK // NW) * NW, NBLK // NW + 1, NBLK // NW)

    @pl.loop(0, nblk)
    def _blk(i):
        b = w + i * NW
        pltpu.sync_copy(src3d.at[b], idxb)
        descs = [
            pltpu.async_copy(p_hbm.at[idxb.at[j]],
                             rows.at[pl.ds(j * 128, 128)], sem)
            for j in range(4)
        ]
        for d_ in descs:
            d_.wait()
        pltpu.sync_copy(rows, out_hbm.at[pl.ds(b * EB, EB)])


def _sc_zero_shared(s, rows, zs, s_sh, ctx_sh):
    z16 = _Z16F()

    @pl.loop(0, 8)
    def _z1(i):
        zs[pl.ds(i * 16, 16)] = z16

    @pl.loop(0, 128)
    def _z2(i):
        for q in range(D // 16):
            rows[i, pl.ds(q * 16, 16)] = z16

    for k in range(5):
        pltpu.sync_copy(zs, s_sh.at[pl.ds(s * 640 + k * 128, 128)])
        pltpu.sync_copy(rows, ctx_sh.at[pl.ds(s * 640 + k * 128, 128)])


def _sc_scale_rows(rows, sc):
    @pl.loop(0, 8)
    def _mul(g):
        a16 = sc[pl.ds(g * 16, 16)]
        for t in range(16):
            av = jnp.full((16,), a16[t], jnp.float32)
            j = g * 16 + t
            for q in range(D // 16):
                rows[j, pl.ds(q * 16, 16)] = rows[j, pl.ds(q * 16, 16)] * av


def _nblocks2(c, s):
    return jnp.where(c < 1, 20, jnp.where(s < 1, 20, 19))


def _sc_writeback(c, s, s_sh, ctx_sh, ctx_out, s_out):
    pltpu.sync_copy(ctx_sh.at[pl.ds(s * 640, 640)],
                    ctx_out.at[c, pl.ds(s * 640, 640)])
    pltpu.sync_copy(s_sh.at[pl.ds(s * 640, 640)], s_out.at[c, s, 0])


def _sc_ctx_getcontext(dst3d, sd_hbm, lcol3d, m_hbm, ctx_out, s_out,
                       idxd, lbuf, tabA, rows, elch, zs,
                       s_sh, ctx_sh, sem):
    c = lax.axis_index("c")
    s = lax.axis_index("s")
    pltpu.sync_copy(sd_hbm, tabA)
    _sc_zero_shared(s, rows, zs, s_sh, ctx_sh)
    plsc.subcore_barrier()

    @pl.loop(0, _nblocks2(c, s))
    def _main(ii):
        i = c + 2 * ii
        b = s + i * NS
        pltpu.sync_copy(dst3d.at[b], idxd)
        pltpu.sync_copy(lcol3d.at[b], lbuf)
        for j in range(4):
            g = pltpu.async_copy(m_hbm.at[pl.ds(b * EB + j * 128, 128)],
                                 rows, sem)
            for k in range(8):
                d16 = idxd[j, pl.ds(k * 16, 16)]
                x = plsc.load_gather(tabA, [d16]) + lbuf[j, pl.ds(k * 16, 16)]
                elch[pl.ds(k * 16, 16)] = jnp.exp(jnp.maximum(x, 0.01 * x))
            pltpu.sync_copy(elch, s_sh.at[idxd.at[j]], add=True)
            g.wait()
            _sc_scale_rows(rows, elch)
            pltpu.sync_copy(rows, ctx_sh.at[idxd.at[j]], add=True)

    plsc.subcore_barrier()
    _sc_writeback(c, s, s_sh, ctx_sh, ctx_out, s_out)


def _sc_ctx_layer(dst3d, src3d, sa_hbm, sb_hbm, hvp_hbm, ctx_out, s_out,
                  idxd, idxs, tabA, tabB, rows, elch, zs,
                  s_sh, ctx_sh, sem):
    c = lax.axis_index("c")
    s = lax.axis_index("s")
    pltpu.sync_copy(sa_hbm, tabA)
    pltpu.sync_copy(sb_hbm, tabB)
    _sc_zero_shared(s, rows, zs, s_sh, ctx_sh)
    plsc.subcore_barrier()

    @pl.loop(0, _nblocks2(c, s))
    def _main(ii):
        i = c + 2 * ii
        b = s + i * NS
        pltpu.sync_copy(dst3d.at[b], idxd)
        pltpu.sync_copy(src3d.at[b], idxs)
        for j in range(4):
            g = pltpu.async_copy(hvp_hbm.at[idxs.at[j]], rows, sem)
            for k in range(8):
                d16 = idxd[j, pl.ds(k * 16, 16)]
                s16 = idxs[j, pl.ds(k * 16, 16)]
                x = (plsc.load_gather(tabA, [d16]) +
                     plsc.load_gather(tabB, [s16]))
                elch[pl.ds(k * 16, 16)] = jnp.exp(jnp.maximum(x, 0.01 * x))
            pltpu.sync_copy(elch, s_sh.at[idxd.at[j]], add=True)
            g.wait()
            _sc_scale_rows(rows, elch)
            pltpu.sync_copy(rows, ctx_sh.at[idxd.at[j]], add=True)

    plsc.subcore_barrier()
    _sc_writeback(c, s, s_sh, ctx_sh, ctx_out, s_out)


_CTX_SCRATCH_COMMON = [
    pltpu.VMEM((128, D), jnp.float32),
    pltpu.VMEM((128,), jnp.float32),
    pltpu.VMEM((128,), jnp.float32),
    pltpu.VMEM_SHARED((NP,), jnp.float32),
    pltpu.VMEM_SHARED((NP, D), jnp.float32),
    pltpu.SemaphoreType.DMA,
]


def _run_sc_gather(p, src3d):
    return pl.kernel(
        _sc_gather_body,
        out_type=jax.ShapeDtypeStruct((E, D), jnp.float32),
        mesh=_sc_mesh(),
        compiler_params=pltpu.CompilerParams(needs_layout_passes=False),
        scratch_types=[
            pltpu.VMEM((4, 128), jnp.int32),
            pltpu.VMEM((EB, D), jnp.float32),
            pltpu.SemaphoreType.DMA,
        ],
    )(p, src3d)


def _run_sc_getcontext(dst3d, sd, lcol3d, m):
    return pl.kernel(
        _sc_ctx_getcontext,
        out_type=[
            jax.ShapeDtypeStruct((NC, NP, D), jnp.float32),
            jax.ShapeDtypeStruct((NC, NS, 1, 640), jnp.float32),
        ],
        mesh=_sc_mesh(),
        compiler_params=pltpu.CompilerParams(needs_layout_passes=False),
        scratch_types=[
            pltpu.VMEM((4, 128), jnp.int32),
            pltpu.VMEM((4, 128), jnp.float32),
            pltpu.VMEM((NP,), jnp.float32),
        ] + _CTX_SCRATCH_COMMON,
    )(dst3d, sd, lcol3d, m)


def _run_sc_layer(dst3d, src3d, sa, sb, hvp):
    return pl.kernel(
        _sc_ctx_layer,
        out_type=[
            jax.ShapeDtypeStruct((NC, NP, D), jnp.float32),
            jax.ShapeDtypeStruct((NC, NS, 1, 640), jnp.float32),
        ],
        mesh=_sc_mesh(),
        compiler_params=pltpu.CompilerParams(needs_layout_passes=False),
        scratch_types=[
            pltpu.VMEM((4, 128), jnp.int32),
            pltpu.VMEM((4, 128), jnp.int32),
            pltpu.VMEM((NP,), jnp.float32),
            pltpu.VMEM((NP,), jnp.float32),
        ] + _CTX_SCRATCH_COMMON,
    )(dst3d, src3d, sa, sb, hvp)


def _full(shape):
    return pl.BlockSpec(shape, lambda *_: tuple(0 for _ in shape))


def kernel(n_feats, e_feats, edge_index, node_graph_ids, params):
    gc = params['gc']
    ei = edge_index.astype(jnp.int32)
    src3d = ei[0].reshape(NBLK, 4, 128)
    dst3d = ei[1].reshape(NBLK, 4, 128)
    nf_p = jnp.pad(n_feats, ((0, NP - N), (0, 0)))
    gid_p = jnp.pad(node_graph_ids.astype(jnp.int32), (0, NP - N),
                    constant_values=G).reshape(NP, 1)

    wpnT = gc['W_pn'].T
    bpn = gc['b_pn'].reshape(1, D)
    wnodeT = gc['W_pe1'][:, :NF].T
    weT = gc['W_pe1'][:, NF:].T
    bpe1 = gc['b_pe1'].reshape(1, D)
    wa0 = gc['W_pe2'][0, :D].reshape(D, 1)
    wbr0 = gc['W_pe2'][0, D:].reshape(1, D)
    b20 = gc['b_pe2'].reshape(1, 1)
    wetT = gc['W_et'].T
    bet = gc['b_et'].reshape(1, D)

    hv_new, p_tab, sd = pl.pallas_call(
        _tc_node_prep,
        grid=(NP // NBN,),
        in_specs=[
            pl.BlockSpec((NBN, D), lambda i: (i, 0)),
            _full((D, D)), _full((1, D)), _full((D, D)), _full((D, 1)),
            _full((1, 1)),
        ],
        out_specs=[
            pl.BlockSpec((NBN, D), lambda i: (i, 0)),
            pl.BlockSpec((NBN, D), lambda i: (i, 0)),
            pl.BlockSpec((NBN, 1), lambda i: (i, 0)),
        ],
        out_shape=[
            jax.ShapeDtypeStruct((NP, D), jnp.float32),
            jax.ShapeDtypeStruct((NP, D), jnp.float32),
            jax.ShapeDtypeStruct((NP, 1), jnp.float32),
        ],
    )(nf_p, wpnT, bpn, wnodeT, wa0, b20)

    psrc = _run_sc_gather(p_tab, src3d)

    m_mat, lcol3 = pl.pallas_call(
        _tc_edge_mm,
        grid=(E // EBM,),
        in_specs=[
            pl.BlockSpec((EBM, D), lambda i: (i, 0)),
            pl.BlockSpec((EBM, EF), lambda i: (i, 0)),
            _full((EF, D)), _full((1, D)),
            _full((D, D)), _full((1, D)), _full((1, D)),
        ],
        out_specs=[
            pl.BlockSpec((EBM, D), lambda i: (i, 0)),
            pl.BlockSpec((1, 1, EBM), lambda i: (i, 0, 0)),
        ],
        out_shape=[
            jax.ShapeDtypeStruct((E, D), jnp.float32),
            jax.ShapeDtypeStruct((E // EBM, 1, EBM), jnp.float32),
        ],
    )(psrc, e_feats, weT, bpe1, wetT, bet, wbr0)
    lcol3d = lcol3.reshape(NBLK, 4, 128)

    ctx, sden4 = _run_sc_getcontext(dst3d, sd.reshape(NP), lcol3d, m_mat)
    sden = sden4.reshape(NC, NP, 1)

    h = hv_new
    for li, lp in enumerate(params['layers']):
        wa = lp['W_pe'][0, :D].reshape(D, 1)
        wb = lp['W_pe'][0, D:].reshape(D, 1)
        bpe = lp['b_pe'].reshape(1, 1)
        gp = params['gc'] if li == 0 else params['layers'][li - 1]
        nf, sa, sb, hvp = pl.pallas_call(
            _tc_gru_prep,
            grid=(NP // NBN,),
            in_specs=[
                pl.BlockSpec((NC, NBN, D), lambda i: (0, i, 0)),
                pl.BlockSpec((NC, NBN, 1), lambda i: (0, i, 0)),
                pl.BlockSpec((NBN, D), lambda i: (i, 0)),
                _full((D, 3 * D)), _full((D, 3 * D)),
                _full((1, 3 * D)), _full((1, 3 * D)),
                _full((D, 1)), _full((D, 1)), _full((1, 1)),
                _full((D, D)), _full((1, D)),
            ],
            out_specs=[
                pl.BlockSpec((NBN, D), lambda i: (i, 0)),
                pl.BlockSpec((NBN, 1), lambda i: (i, 0)),
                pl.BlockSpec((NBN, 1), lambda i: (i, 0)),
                pl.BlockSpec((NBN, D), lambda i: (i, 0)),
            ],
            out_shape=[
                jax.ShapeDtypeStruct((NP, D), jnp.float32),
                jax.ShapeDtypeStruct((NP, 1), jnp.float32),
                jax.ShapeDtypeStruct((NP, 1), jnp.float32),
                jax.ShapeDtypeStruct((NP, D), jnp.float32),
            ],
        )(ctx, sden, h, gp['W_ih'].T, gp['W_hh'].T,
          gp['b_ih'].reshape(1, 3 * D), gp['b_hh'].reshape(1, 3 * D),
          wa, wb, bpe, lp['W_pn'].T, lp['b_pn'].reshape(1, D))
        ctx, sden4 = _run_sc_layer(dst3d, src3d, sa.reshape(NP), sb.reshape(NP), hvp)
        sden = sden4.reshape(NC, NP, 1)
        h = nf

    lp_last = params['layers'][-1]
    nf_final = pl.pallas_call(
        _tc_gru_final,
        grid=(NP // NBN,),
        in_specs=[
            pl.BlockSpec((NC, NBN, D), lambda i: (0, i, 0)),
            pl.BlockSpec((NC, NBN, 1), lambda i: (0, i, 0)),
            pl.BlockSpec((NBN, D), lambda i: (i, 0)),
            _full((D, 3 * D)), _full((D, 3 * D)),
            _full((1, 3 * D)), _full((1, 3 * D)),
        ],
        out_specs=pl.BlockSpec((NBN, D), lambda i: (i, 0)),
        out_shape=jax.ShapeDtypeStruct((NP, D), jnp.float32),
    )(ctx, sden, h, lp_last['W_ih'].T, lp_last['W_hh'].T,
      lp_last['b_ih'].reshape(1, 3 * D), lp_last['b_hh'].reshape(1, 3 * D))

    ro_args = []
    for rp in params['readouts']:
        ro_args += [
            rp['W_pj'].T, rp['b_pj'].reshape(1, D),
            rp['W_cl'][0, :D].reshape(D, 1),
            rp['W_cl'][0, D:].reshape(D, 1),
            rp['b_cl'].reshape(1, 1),
            rp['W_ih'].T, rp['W_hh'].T,
            rp['b_ih'].reshape(1, 3 * D), rp['b_hh'].reshape(1, 3 * D),
        ]

    ro_specs = [
        _full((D, D)), _full((1, D)), _full((D, 1)), _full((D, 1)),
        _full((1, 1)),
        _full((D, 3 * D)), _full((D, 3 * D)), _full((1, 3 * D)),
        _full((1, 3 * D)),
    ]
    g_feats = pl.pallas_call(
        _tc_readout,
        in_specs=[_full((NP, D)), _full((NP, 1))] + ro_specs + ro_specs,
        out_specs=_full((G, D)),
        out_shape=jax.ShapeDtypeStruct((G, D), jnp.float32),
    )(nf_final, gid_p, *ro_args)
    return g_feats


# speedup vs baseline: 16.4294x; 1.0477x over previous
import functools

import jax
import jax.numpy as jnp
from jax import lax
from jax.experimental import pallas as pl
from jax.experimental.pallas import tpu as pltpu
from jax.experimental.pallas import tpu_sc as plsc

N = 10000
NP = 10240
E = 320000
G = 64
NF = 128
EF = 16
D = 128
NC = 2
NS = 16
NW = NC * NS
EB = 512
NBLK = E // EB
ER = E // 128
NSTRIPE = NP // NS
EBM = 2560
NBN = 1024


def _leaky(x):
    return jnp.maximum(x, 0.01 * x)


def _elu(x):
    return jnp.where(x > 0, x, jnp.exp(x) - 1.0)


def _tc_node_prep(x_ref, wpnT, bpn, wnodeT, wa, b2, hv_ref, p_ref, sd_ref):
    x = x_ref[...]
    hv = _leaky(x @ wpnT[...] + bpn[...])
    hv_ref[...] = hv
    p_ref[...] = x @ wnodeT[...]
    sd_ref[...] = hv @ wa[...] + b2[...]


def _tc_edge_mm(psrc_ref, ef_ref, weT, bpe1, wetT, bet, wbr, m_ref, l_ref):
    he = _leaky(psrc_ref[...] + ef_ref[...] @ weT[...] + bpe1[...])
    m_ref[...] = he @ wetT[...] + bet[...]
    l_ref[0, 0, :] = jnp.sum(he * wbr[...], axis=1)


def _gru_block(x, h, wihT, whhT, bih, bhh):
    gi = x @ wihT + bih
    gh = h @ whhT + bhh
    r = jax.nn.sigmoid(gi[:, :D] + gh[:, :D])
    z = jax.nn.sigmoid(gi[:, D:2 * D] + gh[:, D:2 * D])
    n = jnp.tanh(gi[:, 2 * D:] + r * gh[:, 2 * D:])
    return (1.0 - z) * n + z * h


def _tc_gru_prep(ctx_ref, sden_ref, h_ref, wihT, whhT, bih, bhh, wa, wb, bpe,
                 wpnT, bpn, nf_ref, sa_ref, sb_ref, hvp_ref):
    x = _elu((ctx_ref[0] + ctx_ref[1]) /
             (sden_ref[0] + sden_ref[1] + 1e-12))
    nf = jnp.maximum(_gru_block(x, h_ref[...], wihT[...], whhT[...],
                                bih[...], bhh[...]), 0.0)
    nf_ref[...] = nf
    sa_ref[...] = nf @ wa[...] + bpe[...]
    sb_ref[...] = nf @ wb[...]
    hvp_ref[...] = nf @ wpnT[...] + bpn[...]


def _tc_gru_final(ctx_ref, sden_ref, h_ref, wihT, whhT, bih, bhh, nf_ref):
    x = _elu((ctx_ref[0] + ctx_ref[1]) /
             (sden_ref[0] + sden_ref[1] + 1e-12))
    nf_ref[...] = jnp.maximum(
        _gru_block(x, h_ref[...], wihT[...], whhT[...], bih[...], bhh[...]),
        0.0)


def _tc_readout(nf_ref, gid_ref, *refs):
    (wpj0, bpj0, wacl0, wbcl0, bcl0, wih0, whh0, bih0, bhh0,
     wpj1, bpj1, wacl1, wbcl1, bcl1, wih1, whh1, bih1, bhh1, o_ref) = refs
    x = nf_ref[...]
    onehot = (gid_ref[...] == lax.broadcasted_iota(jnp.int32, (NP, G), 1)
              ).astype(jnp.float32)
    cdims = (((0,), (0,)), ((), ()))
    g = lax.dot_general(onehot, x, cdims)
    for (wpj, bpj, wacl, wbcl, bcl, wih, whh, bih, bhh) in (
            (wpj0, bpj0, wacl0, wbcl0, bcl0, wih0, whh0, bih0, bhh0),
            (wpj1, bpj1, wacl1, wbcl1, bcl1, wih1, whh1, bih1, bhh1)):
        ga = jnp.maximum(g, 0.0) @ wacl[...] + bcl[...]
        nb = x @ wbcl[...]
        z = _leaky(onehot @ ga + nb)
        e = jnp.exp(z)
        ssum = lax.dot_general(onehot, e, cdims)
        aa = e / (onehot @ ssum + 1e-12)
        hv = x @ wpj[...] + bpj[...]
        grp = _elu(lax.dot_general(onehot, aa * hv, cdims))
        g = _gru_block(grp, g, wih[...], whh[...], bih[...], bhh[...])
    o_ref[...] = g


def _sc_mesh():
    return plsc.VectorSubcoreMesh(
        core_axis_name="c", subcore_axis_name="s",
        num_cores=NC, num_subcores=NS)


_Z16F = functools.partial(jnp.zeros, (16,), jnp.float32)
_Z16I = functools.partial(jnp.zeros, (16,), jnp.int32)


def _nblocks(s):
    return jnp.where(s < 1, NBLK // NS + 1, NBLK // NS)


def _sc_gather_body(p_hbm, src3d, out_hbm, idxb, rows, sem):
    c = lax.axis_index("c")
    s = lax.axis_index("s")
    w = s * NC + c
    nblk = jnp.where(w < NBLK - (NBLK // NW) * NW, NBLK // NW + 1, NBLK // NW)

    @pl.loop(0, nblk)
    def _blk(i):
        b = w + i * NW
        pltpu.sync_copy(src3d.at[b], idxb)
        descs = [
            pltpu.async_copy(p_hbm.at[idxb.at[j]],
                             rows.at[pl.ds(j * 128, 128)], sem)
            for j in range(4)
        ]
        for d_ in descs:
            d_.wait()
        pltpu.sync_copy(rows, out_hbm.at[pl.ds(b * EB, EB)])


def _sc_zero_shared(s, rows, zs, s_sh, ctx_sh):
    z16 = _Z16F()

    @pl.loop(0, 8)
    def _z1(i):
        zs[pl.ds(i * 16, 16)] = z16

    @pl.loop(0, 128)
    def _z2(i):
        for q in range(D // 16):
            rows[i, pl.ds(q * 16, 16)] = z16

    for k in range(5):
        pltpu.sync_copy(zs, s_sh.at[pl.ds(s * 640 + k * 128, 128)])
        pltpu.sync_copy(rows, ctx_sh.at[pl.ds(s * 640 + k * 128, 128)])


def _sc_scale_rows(rows, sc):
    @pl.loop(0, 8)
    def _mul(g):
        a16 = sc[pl.ds(g * 16, 16)]
        for t in range(16):
            av = jnp.full((16,), a16[t], jnp.float32)
            j = g * 16 + t
            for q in range(D // 16):
                rows[j, pl.ds(q * 16, 16)] = rows[j, pl.ds(q * 16, 16)] * av


def _nblocks2(c, s):
    return jnp.where(c < 1, 20, jnp.where(s < 1, 20, 19))


def _sc_writeback(c, s, s_sh, ctx_sh, ctx_out, s_out):
    pltpu.sync_copy(ctx_sh.at[pl.ds(s * 640, 640)],
                    ctx_out.at[c, pl.ds(s * 640, 640)])
    pltpu.sync_copy(s_sh.at[pl.ds(s * 640, 640)], s_out.at[c, s, 0])


def _sc_ctx_getcontext(dst3d, sd_hbm, lcol3d, m_hbm, ctx_out, s_out,
                       idxd, lbuf, tabA, rows0, rows1, el0, el1, zs,
                       s_sh, ctx_sh, isem1, isem2, gsem,
                       esem0, esem1, csem0, csem1):
    c = lax.axis_index("c")
    s = lax.axis_index("s")
    pltpu.sync_copy(sd_hbm, tabA)
    _sc_zero_shared(s, rows0, zs, s_sh, ctx_sh)
    plsc.subcore_barrier()
    rowsb = (rows0, rows1)
    elb = (el0, el1)
    esems = (esem0, esem1)
    csems = (csem0, csem1)

    @pl.loop(0, _nblocks2(c, s))
    def _main(ii):
        i = c + 2 * ii
        b = s + i * NS
        di = pltpu.async_copy(dst3d.at[b], idxd, isem1)
        li = pltpu.async_copy(lcol3d.at[b], lbuf, isem2)
        di.wait()
        li.wait()
        epend = [None, None]
        cpend = [None, None]
        for j in range(4):
            p = j % 2
            rows, elch = rowsb[p], elb[p]
            if cpend[p] is not None:
                cpend[p].wait()
            g = pltpu.async_copy(m_hbm.at[pl.ds(b * EB + j * 128, 128)],
                                 rows, gsem)
            if epend[p] is not None:
                epend[p].wait()
            for k in range(8):
                d16 = idxd[j, pl.ds(k * 16, 16)]
                x = plsc.load_gather(tabA, [d16]) + lbuf[j, pl.ds(k * 16, 16)]
                elch[pl.ds(k * 16, 16)] = jnp.exp(jnp.maximum(x, 0.01 * x))
            epend[p] = pltpu.async_copy(elch, s_sh.at[idxd.at[j]], esems[p],
                                        add=True)
            g.wait()
            _sc_scale_rows(rows, elch)
            cpend[p] = pltpu.async_copy(rows, ctx_sh.at[idxd.at[j]], csems[p],
                                        add=True)
        for d_ in epend + cpend:
            d_.wait()

    plsc.subcore_barrier()
    _sc_writeback(c, s, s_sh, ctx_sh, ctx_out, s_out)


def _sc_ctx_layer(dst3d, src3d, sa_hbm, sb_hbm, hvp_hbm, ctx_out, s_out,
                  idxd, idxs, tabA, tabB, rows, el0, el1, zs,
                  s_sh, ctx_sh, isem1, isem2, gsem, esem0, esem1):
    c = lax.axis_index("c")
    s = lax.axis_index("s")
    pltpu.sync_copy(sa_hbm, tabA)
    pltpu.sync_copy(sb_hbm, tabB)
    _sc_zero_shared(s, rows, zs, s_sh, ctx_sh)
    plsc.subcore_barrier()
    elb = (el0, el1)
    esems = (esem0, esem1)

    @pl.loop(0, _nblocks2(c, s))
    def _main(ii):
        i = c + 2 * ii
        b = s + i * NS
        di = pltpu.async_copy(dst3d.at[b], idxd, isem1)
        si = pltpu.async_copy(src3d.at[b], idxs, isem2)
        di.wait()
        si.wait()
        epend = [None, None]
        for j in range(4):
            p = j % 2
            elch = elb[p]
            g = pltpu.async_copy(hvp_hbm.at[idxs.at[j]], rows, gsem)
            if epend[p] is not None:
                epend[p].wait()
            for k in range(8):
                d16 = idxd[j, pl.ds(k * 16, 16)]
                s16 = idxs[j, pl.ds(k * 16, 16)]
                x = (plsc.load_gather(tabA, [d16]) +
                     plsc.load_gather(tabB, [s16]))
                elch[pl.ds(k * 16, 16)] = jnp.exp(jnp.maximum(x, 0.01 * x))
            epend[p] = pltpu.async_copy(elch, s_sh.at[idxd.at[j]], esems[p],
                                        add=True)
            g.wait()
            _sc_scale_rows(rows, elch)
            pltpu.sync_copy(rows, ctx_sh.at[idxd.at[j]], add=True)
        for d_ in epend:
            d_.wait()

    plsc.subcore_barrier()
    _sc_writeback(c, s, s_sh, ctx_sh, ctx_out, s_out)


_CTX_SHARED_SCRATCH = [
    pltpu.VMEM_SHARED((NP,), jnp.float32),
    pltpu.VMEM_SHARED((NP, D), jnp.float32),
]


def _run_sc_gather(p, src3d):
    return pl.kernel(
        _sc_gather_body,
        out_type=jax.ShapeDtypeStruct((E, D), jnp.float32),
        mesh=_sc_mesh(),
        compiler_params=pltpu.CompilerParams(needs_layout_passes=False),
        scratch_types=[
            pltpu.VMEM((4, 128), jnp.int32),
            pltpu.VMEM((EB, D), jnp.float32),
            pltpu.SemaphoreType.DMA,
        ],
    )(p, src3d)


def _run_sc_getcontext(dst3d, sd, lcol3d, m):
    return pl.kernel(
        _sc_ctx_getcontext,
        out_type=[
            jax.ShapeDtypeStruct((NC, NP, D), jnp.float32),
            jax.ShapeDtypeStruct((NC, NS, 1, 640), jnp.float32),
        ],
        mesh=_sc_mesh(),
        compiler_params=pltpu.CompilerParams(needs_layout_passes=False),
        scratch_types=[
            pltpu.VMEM((4, 128), jnp.int32),
            pltpu.VMEM((4, 128), jnp.float32),
            pltpu.VMEM((NP,), jnp.float32),
            pltpu.VMEM((128, D), jnp.float32),
            pltpu.VMEM((128, D), jnp.float32),
            pltpu.VMEM((128,), jnp.float32),
            pltpu.VMEM((128,), jnp.float32),
            pltpu.VMEM((128,), jnp.float32),
        ] + _CTX_SHARED_SCRATCH + [pltpu.SemaphoreType.DMA] * 7,
    )(dst3d, sd, lcol3d, m)


def _run_sc_layer(dst3d, src3d, sa, sb, hvp):
    return pl.kernel(
        _sc_ctx_layer,
        out_type=[
            jax.ShapeDtypeStruct((NC, NP, D), jnp.float32),
            jax.ShapeDtypeStruct((NC, NS, 1, 640), jnp.float32),
        ],
        mesh=_sc_mesh(),
        compiler_params=pltpu.CompilerParams(needs_layout_passes=False),
        scratch_types=[
            pltpu.VMEM((4, 128), jnp.int32),
            pltpu.VMEM((4, 128), jnp.int32),
            pltpu.VMEM((NP,), jnp.float32),
            pltpu.VMEM((NP,), jnp.float32),
            pltpu.VMEM((128, D), jnp.float32),
            pltpu.VMEM((128,), jnp.float32),
            pltpu.VMEM((128,), jnp.float32),
            pltpu.VMEM((128,), jnp.float32),
        ] + _CTX_SHARED_SCRATCH + [pltpu.SemaphoreType.DMA] * 5,
    )(dst3d, src3d, sa, sb, hvp)


def _full(shape):
    return pl.BlockSpec(shape, lambda *_: tuple(0 for _ in shape))


def kernel(n_feats, e_feats, edge_index, node_graph_ids, params):
    gc = params['gc']
    ei = edge_index.astype(jnp.int32)
    src3d = ei[0].reshape(NBLK, 4, 128)
    dst3d = ei[1].reshape(NBLK, 4, 128)
    nf_p = jnp.pad(n_feats, ((0, NP - N), (0, 0)))
    gid_p = jnp.pad(node_graph_ids.astype(jnp.int32), (0, NP - N),
                    constant_values=G).reshape(NP, 1)

    wpnT = gc['W_pn'].T
    bpn = gc['b_pn'].reshape(1, D)
    wnodeT = gc['W_pe1'][:, :NF].T
    weT = gc['W_pe1'][:, NF:].T
    bpe1 = gc['b_pe1'].reshape(1, D)
    wa0 = gc['W_pe2'][0, :D].reshape(D, 1)
    wbr0 = gc['W_pe2'][0, D:].reshape(1, D)
    b20 = gc['b_pe2'].reshape(1, 1)
    wetT = gc['W_et'].T
    bet = gc['b_et'].reshape(1, D)

    hv_new, p_tab, sd = pl.pallas_call(
        _tc_node_prep,
        grid=(NP // NBN,),
        in_specs=[
            pl.BlockSpec((NBN, D), lambda i: (i, 0)),
            _full((D, D)), _full((1, D)), _full((D, D)), _full((D, 1)),
            _full((1, 1)),
        ],
        out_specs=[
            pl.BlockSpec((NBN, D), lambda i: (i, 0)),
            pl.BlockSpec((NBN, D), lambda i: (i, 0)),
            pl.BlockSpec((NBN, 1), lambda i: (i, 0)),
        ],
        out_shape=[
            jax.ShapeDtypeStruct((NP, D), jnp.float32),
            jax.ShapeDtypeStruct((NP, D), jnp.float32),
            jax.ShapeDtypeStruct((NP, 1), jnp.float32),
        ],
    )(nf_p, wpnT, bpn, wnodeT, wa0, b20)

    psrc = _run_sc_gather(p_tab, src3d)

    m_mat, lcol3 = pl.pallas_call(
        _tc_edge_mm,
        grid=(E // EBM,),
        in_specs=[
            pl.BlockSpec((EBM, D), lambda i: (i, 0)),
            pl.BlockSpec((EBM, EF), lambda i: (i, 0)),
            _full((EF, D)), _full((1, D)),
            _full((D, D)), _full((1, D)), _full((1, D)),
        ],
        out_specs=[
            pl.BlockSpec((EBM, D), lambda i: (i, 0)),
            pl.BlockSpec((1, 1, EBM), lambda i: (i, 0, 0)),
        ],
        out_shape=[
            jax.ShapeDtypeStruct((E, D), jnp.float32),
            jax.ShapeDtypeStruct((E // EBM, 1, EBM), jnp.float32),
        ],
    )(psrc, e_feats, weT, bpe1, wetT, bet, wbr0)
    lcol3d = lcol3.reshape(NBLK, 4, 128)

    ctx, sden4 = _run_sc_getcontext(dst3d, sd.reshape(NP), lcol3d, m_mat)
    sden = sden4.reshape(NC, NP, 1)

    h = hv_new
    for li, lp in enumerate(params['layers']):
        wa = lp['W_pe'][0, :D].reshape(D, 1)
        wb = lp['W_pe'][0, D:].reshape(D, 1)
        bpe = lp['b_pe'].reshape(1, 1)
        gp = params['gc'] if li == 0 else params['layers'][li - 1]
        nf, sa, sb, hvp = pl.pallas_call(
            _tc_gru_prep,
            grid=(NP // NBN,),
            in_specs=[
                pl.BlockSpec((NC, NBN, D), lambda i: (0, i, 0)),
                pl.BlockSpec((NC, NBN, 1), lambda i: (0, i, 0)),
                pl.BlockSpec((NBN, D), lambda i: (i, 0)),
                _full((D, 3 * D)), _full((D, 3 * D)),
                _full((1, 3 * D)), _full((1, 3 * D)),
                _full((D, 1)), _full((D, 1)), _full((1, 1)),
                _full((D, D)), _full((1, D)),
            ],
            out_specs=[
                pl.BlockSpec((NBN, D), lambda i: (i, 0)),
                pl.BlockSpec((NBN, 1), lambda i: (i, 0)),
                pl.BlockSpec((NBN, 1), lambda i: (i, 0)),
                pl.BlockSpec((NBN, D), lambda i: (i, 0)),
            ],
            out_shape=[
                jax.ShapeDtypeStruct((NP, D), jnp.float32),
                jax.ShapeDtypeStruct((NP, 1), jnp.float32),
                jax.ShapeDtypeStruct((NP, 1), jnp.float32),
                jax.ShapeDtypeStruct((NP, D), jnp.float32),
            ],
        )(ctx, sden, h, gp['W_ih'].T, gp['W_hh'].T,
          gp['b_ih'].reshape(1, 3 * D), gp['b_hh'].reshape(1, 3 * D),
          wa, wb, bpe, lp['W_pn'].T, lp['b_pn'].reshape(1, D))
        ctx, sden4 = _run_sc_layer(dst3d, src3d, sa.reshape(NP), sb.reshape(NP), hvp)
        sden = sden4.reshape(NC, NP, 1)
        h = nf

    lp_last = params['layers'][-1]
    nf_final = pl.pallas_call(
        _tc_gru_final,
        grid=(NP // NBN,),
        in_specs=[
            pl.BlockSpec((NC, NBN, D), lambda i: (0, i, 0)),
            pl.BlockSpec((NC, NBN, 1), lambda i: (0, i, 0)),
            pl.BlockSpec((NBN, D), lambda i: (i, 0)),
            _full((D, 3 * D)), _full((D, 3 * D)),
            _full((1, 3 * D)), _full((1, 3 * D)),
        ],
        out_specs=pl.BlockSpec((NBN, D), lambda i: (i, 0)),
        out_shape=jax.ShapeDtypeStruct((NP, D), jnp.float32),
    )(ctx, sden, h, lp_last['W_ih'].T, lp_last['W_hh'].T,
      lp_last['b_ih'].reshape(1, 3 * D), lp_last['b_hh'].reshape(1, 3 * D))

    ro_args = []
    for rp in params['readouts']:
        ro_args += [
            rp['W_pj'].T, rp['b_pj'].reshape(1, D),
            rp['W_cl'][0, :D].reshape(D, 1),
            rp['W_cl'][0, D:].reshape(D, 1),
            rp['b_cl'].reshape(1, 1),
            rp['W_ih'].T, rp['W_hh'].T,
            rp['b_ih'].reshape(1, 3 * D), rp['b_hh'].reshape(1, 3 * D),
        ]

    ro_specs = [
        _full((D, D)), _full((1, D)), _full((D, 1)), _full((D, 1)),
        _full((1, 1)),
        _full((D, 3 * D)), _full((D, 3 * D)), _full((1, 3 * D)),
        _full((1, 3 * D)),
    ]
    g_feats = pl.pallas_call(
        _tc_readout,
        in_specs=[_full((NP, D)), _full((NP, 1))] + ro_specs + ro_specs,
        out_specs=_full((G, D)),
        out_shape=jax.ShapeDtypeStruct((G, D), jnp.float32),
    )(nf_final, gid_p, *ro_args)
    return g_feats


# speedup vs baseline: 16.4529x; 1.0014x over previous
import functools

import jax
import jax.numpy as jnp
from jax import lax
from jax.experimental import pallas as pl
from jax.experimental.pallas import tpu as pltpu
from jax.experimental.pallas import tpu_sc as plsc

N = 10000
NP = 10240
E = 320000
G = 64
NF = 128
EF = 16
D = 128
NC = 2
NS = 16
NW = NC * NS
EB = 512
NBLK = E // EB
ER = E // 128
NSTRIPE = NP // NS
EBM = 2560
NBN = 1024


def _leaky(x):
    return jnp.maximum(x, 0.01 * x)


def _elu(x):
    return jnp.where(x > 0, x, jnp.exp(x) - 1.0)


def _tc_node_prep(x_ref, wpnT, bpn, wnodeT, wa, b2, hv_ref, p_ref, sd_ref):
    x = x_ref[...]
    hv = _leaky(x @ wpnT[...] + bpn[...])
    hv_ref[...] = hv
    p_ref[...] = x @ wnodeT[...]
    sd_ref[...] = hv @ wa[...] + b2[...]


def _tc_edge_mm(psrc_ref, ef_ref, weT, bpe1, wetT, bet, wbr, m_ref, l_ref):
    he = _leaky(psrc_ref[...] + ef_ref[...] @ weT[...] + bpe1[...])
    m_ref[...] = he @ wetT[...] + bet[...]
    l_ref[0, 0, :] = jnp.sum(he * wbr[...], axis=1)


def _gru_block(x, h, wihT, whhT, bih, bhh):
    gi = x @ wihT + bih
    gh = h @ whhT + bhh
    r = jax.nn.sigmoid(gi[:, :D] + gh[:, :D])
    z = jax.nn.sigmoid(gi[:, D:2 * D] + gh[:, D:2 * D])
    n = jnp.tanh(gi[:, 2 * D:] + r * gh[:, 2 * D:])
    return (1.0 - z) * n + z * h


def _tc_gru_prep(ctx_ref, sden_ref, h_ref, wihT, whhT, bih, bhh, wa, wb, bpe,
                 wpnT, bpn, nf_ref, sa_ref, sb_ref, hvp_ref):
    x = _elu((ctx_ref[0] + ctx_ref[1]) /
             (sden_ref[0] + sden_ref[1] + 1e-12))
    nf = jnp.maximum(_gru_block(x, h_ref[...], wihT[...], whhT[...],
                                bih[...], bhh[...]), 0.0)
    nf_ref[...] = nf
    sa_ref[...] = nf @ wa[...] + bpe[...]
    sb_ref[...] = nf @ wb[...]
    hvp_ref[...] = nf @ wpnT[...] + bpn[...]


def _tc_gru_final(ctx_ref, sden_ref, h_ref, wihT, whhT, bih, bhh, nf_ref):
    x = _elu((ctx_ref[0] + ctx_ref[1]) /
             (sden_ref[0] + sden_ref[1] + 1e-12))
    nf_ref[...] = jnp.maximum(
        _gru_block(x, h_ref[...], wihT[...], whhT[...], bih[...], bhh[...]),
        0.0)


def _tc_readout(nf_ref, gid_ref, *refs):
    (wpj0, bpj0, wacl0, wbcl0, bcl0, wih0, whh0, bih0, bhh0,
     wpj1, bpj1, wacl1, wbcl1, bcl1, wih1, whh1, bih1, bhh1, o_ref) = refs
    x = nf_ref[...]
    onehot = (gid_ref[...] == lax.broadcasted_iota(jnp.int32, (NP, G), 1)
              ).astype(jnp.float32)
    cdims = (((0,), (0,)), ((), ()))
    g = lax.dot_general(onehot, x, cdims)
    for (wpj, bpj, wacl, wbcl, bcl, wih, whh, bih, bhh) in (
            (wpj0, bpj0, wacl0, wbcl0, bcl0, wih0, whh0, bih0, bhh0),
            (wpj1, bpj1, wacl1, wbcl1, bcl1, wih1, whh1, bih1, bhh1)):
        ga = jnp.maximum(g, 0.0) @ wacl[...] + bcl[...]
        nb = x @ wbcl[...]
        z = _leaky(onehot @ ga + nb)
        e = jnp.exp(z)
        ssum = lax.dot_general(onehot, e, cdims)
        aa = e / (onehot @ ssum + 1e-12)
        hv = x @ wpj[...] + bpj[...]
        grp = _elu(lax.dot_general(onehot, aa * hv, cdims))
        g = _gru_block(grp, g, wih[...], whh[...], bih[...], bhh[...])
    o_ref[...] = g


def _sc_mesh():
    return plsc.VectorSubcoreMesh(
        core_axis_name="c", subcore_axis_name="s",
        num_cores=NC, num_subcores=NS)


_Z16F = functools.partial(jnp.zeros, (16,), jnp.float32)
_Z16I = functools.partial(jnp.zeros, (16,), jnp.int32)


def _nblocks(s):
    return jnp.where(s < 1, NBLK // NS + 1, NBLK // NS)


def _sc_gather_body(p_hbm, src3d, out_hbm, idxb, buf0, buf1, gsem, wsem0,
                    wsem1):
    c = lax.axis_index("c")
    s = lax.axis_index("s")
    w = s * NC + c
    nblk = jnp.where(w < NBLK - (NBLK // NW) * NW, NBLK // NW + 1, NBLK // NW)
    bufs = (buf0, buf1)
    wsems = (wsem0, wsem1)

    @pl.loop(0, nblk)
    def _blk(i):
        b = w + i * NW
        pltpu.sync_copy(src3d.at[b], idxb)
        wpend = [None, None]
        for h in range(2):
            buf = bufs[h]
            descs = [
                pltpu.async_copy(p_hbm.at[idxb.at[2 * h + j]],
                                 buf.at[pl.ds(j * 128, 128)], gsem)
                for j in range(2)
            ]
            for d_ in descs:
                d_.wait()
            wpend[h] = pltpu.async_copy(
                buf, out_hbm.at[pl.ds(b * EB + h * 256, 256)], wsems[h])
        wpend[0].wait()
        wpend[1].wait()


def _sc_zero_shared(s, rows, zs, s_sh, ctx_sh):
    z16 = _Z16F()

    @pl.loop(0, 8)
    def _z1(i):
        zs[pl.ds(i * 16, 16)] = z16

    @pl.loop(0, 128)
    def _z2(i):
        for q in range(D // 16):
            rows[i, pl.ds(q * 16, 16)] = z16

    for k in range(5):
        pltpu.sync_copy(zs, s_sh.at[pl.ds(s * 640 + k * 128, 128)])
        pltpu.sync_copy(rows, ctx_sh.at[pl.ds(s * 640 + k * 128, 128)])


def _sc_scale_rows(rows, sc):
    @pl.loop(0, 8)
    def _mul(g):
        a16 = sc[pl.ds(g * 16, 16)]
        for t in range(16):
            av = jnp.full((16,), a16[t], jnp.float32)
            j = g * 16 + t
            for q in range(D // 16):
                rows[j, pl.ds(q * 16, 16)] = rows[j, pl.ds(q * 16, 16)] * av


def _nblocks2(c, s):
    return jnp.where(c < 1, 20, jnp.where(s < 1, 20, 19))


def _sc_writeback(c, s, s_sh, ctx_sh, ctx_out, s_out):
    pltpu.sync_copy(ctx_sh.at[pl.ds(s * 640, 640)],
                    ctx_out.at[c, pl.ds(s * 640, 640)])
    pltpu.sync_copy(s_sh.at[pl.ds(s * 640, 640)], s_out.at[c, s, 0])


def _sc_ctx_getcontext(dst3d, sd_hbm, lcol3d, m_hbm, ctx_out, s_out,
                       idxd, lbuf, tabA, rows0, rows1, el0, el1, zs,
                       s_sh, ctx_sh, isem1, isem2, gsem,
                       esem0, esem1, csem0, csem1):
    c = lax.axis_index("c")
    s = lax.axis_index("s")
    pltpu.sync_copy(sd_hbm, tabA)
    _sc_zero_shared(s, rows0, zs, s_sh, ctx_sh)
    plsc.subcore_barrier()
    rowsb = (rows0, rows1)
    elb = (el0, el1)
    esems = (esem0, esem1)
    csems = (csem0, csem1)

    @pl.loop(0, _nblocks2(c, s))
    def _main(ii):
        i = c + 2 * ii
        b = s + i * NS
        di = pltpu.async_copy(dst3d.at[b], idxd, isem1)
        li = pltpu.async_copy(lcol3d.at[b], lbuf, isem2)
        di.wait()
        li.wait()
        epend = [None, None]
        cpend = [None, None]
        for j in range(4):
            p = j % 2
            rows, elch = rowsb[p], elb[p]
            if cpend[p] is not None:
                cpend[p].wait()
            g = pltpu.async_copy(m_hbm.at[pl.ds(b * EB + j * 128, 128)],
                                 rows, gsem)
            if epend[p] is not None:
                epend[p].wait()
            for k in range(8):
                d16 = idxd[j, pl.ds(k * 16, 16)]
                x = plsc.load_gather(tabA, [d16]) + lbuf[j, pl.ds(k * 16, 16)]
                elch[pl.ds(k * 16, 16)] = jnp.exp(jnp.maximum(x, 0.01 * x))
            epend[p] = pltpu.async_copy(elch, s_sh.at[idxd.at[j]], esems[p],
                                        add=True)
            g.wait()
            _sc_scale_rows(rows, elch)
            cpend[p] = pltpu.async_copy(rows, ctx_sh.at[idxd.at[j]], csems[p],
                                        add=True)
        for d_ in epend + cpend:
            d_.wait()

    plsc.subcore_barrier()
    _sc_writeback(c, s, s_sh, ctx_sh, ctx_out, s_out)


def _sc_ctx_layer(dst3d, src3d, sa_hbm, sb_hbm, hvp_hbm, ctx_out, s_out,
                  idxd, idxs, tabA, tabB, rows, el0, el1, zs,
                  s_sh, ctx_sh, isem1, isem2, gsem, esem0, esem1):
    c = lax.axis_index("c")
    s = lax.axis_index("s")
    pltpu.sync_copy(sa_hbm, tabA)
    pltpu.sync_copy(sb_hbm, tabB)
    _sc_zero_shared(s, rows, zs, s_sh, ctx_sh)
    plsc.subcore_barrier()
    elb = (el0, el1)
    esems = (esem0, esem1)

    @pl.loop(0, _nblocks2(c, s))
    def _main(ii):
        i = c + 2 * ii
        b = s + i * NS
        di = pltpu.async_copy(dst3d.at[b], idxd, isem1)
        si = pltpu.async_copy(src3d.at[b], idxs, isem2)
        di.wait()
        si.wait()
        epend = [None, None]
        for j in range(4):
            p = j % 2
            elch = elb[p]
            g = pltpu.async_copy(hvp_hbm.at[idxs.at[j]], rows, gsem)
            if epend[p] is not None:
                epend[p].wait()
            for k in range(8):
                d16 = idxd[j, pl.ds(k * 16, 16)]
                s16 = idxs[j, pl.ds(k * 16, 16)]
                x = (plsc.load_gather(tabA, [d16]) +
                     plsc.load_gather(tabB, [s16]))
                elch[pl.ds(k * 16, 16)] = jnp.exp(jnp.maximum(x, 0.01 * x))
            epend[p] = pltpu.async_copy(elch, s_sh.at[idxd.at[j]], esems[p],
                                        add=True)
            g.wait()
            _sc_scale_rows(rows, elch)
            pltpu.sync_copy(rows, ctx_sh.at[idxd.at[j]], add=True)
        for d_ in epend:
            d_.wait()

    plsc.subcore_barrier()
    _sc_writeback(c, s, s_sh, ctx_sh, ctx_out, s_out)


_CTX_SHARED_SCRATCH = [
    pltpu.VMEM_SHARED((NP,), jnp.float32),
    pltpu.VMEM_SHARED((NP, D), jnp.float32),
]


def _run_sc_gather(p, src3d):
    return pl.kernel(
        _sc_gather_body,
        out_type=jax.ShapeDtypeStruct((E, D), jnp.float32),
        mesh=_sc_mesh(),
        compiler_params=pltpu.CompilerParams(needs_layout_passes=False),
        scratch_types=[
            pltpu.VMEM((4, 128), jnp.int32),
            pltpu.VMEM((256, D), jnp.float32),
            pltpu.VMEM((256, D), jnp.float32),
            pltpu.SemaphoreType.DMA,
            pltpu.SemaphoreType.DMA,
            pltpu.SemaphoreType.DMA,
        ],
    )(p, src3d)


def _run_sc_getcontext(dst3d, sd, lcol3d, m):
    return pl.kernel(
        _sc_ctx_getcontext,
        out_type=[
            jax.ShapeDtypeStruct((NC, NP, D), jnp.float32),
            jax.ShapeDtypeStruct((NC, NS, 1, 640), jnp.float32),
        ],
        mesh=_sc_mesh(),
        compiler_params=pltpu.CompilerParams(needs_layout_passes=False),
        scratch_types=[
            pltpu.VMEM((4, 128), jnp.int32),
            pltpu.VMEM((4, 128), jnp.float32),
            pltpu.VMEM((NP,), jnp.float32),
            pltpu.VMEM((128, D), jnp.float32),
            pltpu.VMEM((128, D), jnp.float32),
            pltpu.VMEM((128,), jnp.float32),
            pltpu.VMEM((128,), jnp.float32),
            pltpu.VMEM((128,), jnp.float32),
        ] + _CTX_SHARED_SCRATCH + [pltpu.SemaphoreType.DMA] * 7,
    )(dst3d, sd, lcol3d, m)


def _run_sc_layer(dst3d, src3d, sa, sb, hvp):
    return pl.kernel(
        _sc_ctx_layer,
        out_type=[
            jax.ShapeDtypeStruct((NC, NP, D), jnp.float32),
            jax.ShapeDtypeStruct((NC, NS, 1, 640), jnp.float32),
        ],
        mesh=_sc_mesh(),
        compiler_params=pltpu.CompilerParams(needs_layout_passes=False),
        scratch_types=[
            pltpu.VMEM((4, 128), jnp.int32),
            pltpu.VMEM((4, 128), jnp.int32),
            pltpu.VMEM((NP,), jnp.float32),
            pltpu.VMEM((NP,), jnp.float32),
            pltpu.VMEM((128, D), jnp.float32),
            pltpu.VMEM((128,), jnp.float32),
            pltpu.VMEM((128,), jnp.float32),
            pltpu.VMEM((128,), jnp.float32),
        ] + _CTX_SHARED_SCRATCH + [pltpu.SemaphoreType.DMA] * 5,
    )(dst3d, src3d, sa, sb, hvp)


def _full(shape):
    return pl.BlockSpec(shape, lambda *_: tuple(0 for _ in shape))


def kernel(n_feats, e_feats, edge_index, node_graph_ids, params):
    gc = params['gc']
    ei = edge_index.astype(jnp.int32)
    src3d = ei[0].reshape(NBLK, 4, 128)
    dst3d = ei[1].reshape(NBLK, 4, 128)
    nf_p = jnp.pad(n_feats, ((0, NP - N), (0, 0)))
    gid_p = jnp.pad(node_graph_ids.astype(jnp.int32), (0, NP - N),
                    constant_values=G).reshape(NP, 1)

    wpnT = gc['W_pn'].T
    bpn = gc['b_pn'].reshape(1, D)
    wnodeT = gc['W_pe1'][:, :NF].T
    weT = gc['W_pe1'][:, NF:].T
    bpe1 = gc['b_pe1'].reshape(1, D)
    wa0 = gc['W_pe2'][0, :D].reshape(D, 1)
    wbr0 = gc['W_pe2'][0, D:].reshape(1, D)
    b20 = gc['b_pe2'].reshape(1, 1)
    wetT = gc['W_et'].T
    bet = gc['b_et'].reshape(1, D)

    hv_new, p_tab, sd = pl.pallas_call(
        _tc_node_prep,
        grid=(NP // NBN,),
        in_specs=[
            pl.BlockSpec((NBN, D), lambda i: (i, 0)),
            _full((D, D)), _full((1, D)), _full((D, D)), _full((D, 1)),
            _full((1, 1)),
        ],
        out_specs=[
            pl.BlockSpec((NBN, D), lambda i: (i, 0)),
            pl.BlockSpec((NBN, D), lambda i: (i, 0)),
            pl.BlockSpec((NBN, 1), lambda i: (i, 0)),
        ],
        out_shape=[
            jax.ShapeDtypeStruct((NP, D), jnp.float32),
            jax.ShapeDtypeStruct((NP, D), jnp.float32),
            jax.ShapeDtypeStruct((NP, 1), jnp.float32),
        ],
    )(nf_p, wpnT, bpn, wnodeT, wa0, b20)

    psrc = _run_sc_gather(p_tab, src3d)

    m_mat, lcol3 = pl.pallas_call(
        _tc_edge_mm,
        grid=(E // EBM,),
        in_specs=[
            pl.BlockSpec((EBM, D), lambda i: (i, 0)),
            pl.BlockSpec((EBM, EF), lambda i: (i, 0)),
            _full((EF, D)), _full((1, D)),
            _full((D, D)), _full((1, D)), _full((1, D)),
        ],
        out_specs=[
            pl.BlockSpec((EBM, D), lambda i: (i, 0)),
            pl.BlockSpec((1, 1, EBM), lambda i: (i, 0, 0)),
        ],
        out_shape=[
            jax.ShapeDtypeStruct((E, D), jnp.float32),
            jax.ShapeDtypeStruct((E // EBM, 1, EBM), jnp.float32),
        ],
    )(psrc, e_feats, weT, bpe1, wetT, bet, wbr0)
    lcol3d = lcol3.reshape(NBLK, 4, 128)

    ctx, sden4 = _run_sc_getcontext(dst3d, sd.reshape(NP), lcol3d, m_mat)
    sden = sden4.reshape(NC, NP, 1)

    h = hv_new
    for li, lp in enumerate(params['layers']):
        wa = lp['W_pe'][0, :D].reshape(D, 1)
        wb = lp['W_pe'][0, D:].reshape(D, 1)
        bpe = lp['b_pe'].reshape(1, 1)
        gp = params['gc'] if li == 0 else params['layers'][li - 1]
        nf, sa, sb, hvp = pl.pallas_call(
            _tc_gru_prep,
            grid=(NP // NBN,),
            in_specs=[
                pl.BlockSpec((NC, NBN, D), lambda i: (0, i, 0)),
                pl.BlockSpec((NC, NBN, 1), lambda i: (0, i, 0)),
                pl.BlockSpec((NBN, D), lambda i: (i, 0)),
                _full((D, 3 * D)), _full((D, 3 * D)),
                _full((1, 3 * D)), _full((1, 3 * D)),
                _full((D, 1)), _full((D, 1)), _full((1, 1)),
                _full((D, D)), _full((1, D)),
            ],
            out_specs=[
                pl.BlockSpec((NBN, D), lambda i: (i, 0)),
                pl.BlockSpec((NBN, 1), lambda i: (i, 0)),
                pl.BlockSpec((NBN, 1), lambda i: (i, 0)),
                pl.BlockSpec((NBN, D), lambda i: (i, 0)),
            ],
            out_shape=[
                jax.ShapeDtypeStruct((NP, D), jnp.float32),
                jax.ShapeDtypeStruct((NP, 1), jnp.float32),
                jax.ShapeDtypeStruct((NP, 1), jnp.float32),
                jax.ShapeDtypeStruct((NP, D), jnp.float32),
            ],
        )(ctx, sden, h, gp['W_ih'].T, gp['W_hh'].T,
          gp['b_ih'].reshape(1, 3 * D), gp['b_hh'].reshape(1, 3 * D),
          wa, wb, bpe, lp['W_pn'].T, lp['b_pn'].reshape(1, D))
        ctx, sden4 = _run_sc_layer(dst3d, src3d, sa.reshape(NP), sb.reshape(NP), hvp)
        sden = sden4.reshape(NC, NP, 1)
        h = nf

    lp_last = params['layers'][-1]
    nf_final = pl.pallas_call(
        _tc_gru_final,
        grid=(NP // NBN,),
        in_specs=[
            pl.BlockSpec((NC, NBN, D), lambda i: (0, i, 0)),
            pl.BlockSpec((NC, NBN, 1), lambda i: (0, i, 0)),
            pl.BlockSpec((NBN, D), lambda i: (i, 0)),
            _full((D, 3 * D)), _full((D, 3 * D)),
            _full((1, 3 * D)), _full((1, 3 * D)),
        ],
        out_specs=pl.BlockSpec((NBN, D), lambda i: (i, 0)),
        out_shape=jax.ShapeDtypeStruct((NP, D), jnp.float32),
    )(ctx, sden, h, lp_last['W_ih'].T, lp_last['W_hh'].T,
      lp_last['b_ih'].reshape(1, 3 * D), lp_last['b_hh'].reshape(1, 3 * D))

    ro_args = []
    for rp in params['readouts']:
        ro_args += [
            rp['W_pj'].T, rp['b_pj'].reshape(1, D),
            rp['W_cl'][0, :D].reshape(D, 1),
            rp['W_cl'][0, D:].reshape(D, 1),
            rp['b_cl'].reshape(1, 1),
            rp['W_ih'].T, rp['W_hh'].T,
            rp['b_ih'].reshape(1, 3 * D), rp['b_hh'].reshape(1, 3 * D),
        ]

    ro_specs = [
        _full((D, D)), _full((1, D)), _full((D, 1)), _full((D, 1)),
        _full((1, 1)),
        _full((D, 3 * D)), _full((D, 3 * D)), _full((1, 3 * D)),
        _full((1, 3 * D)),
    ]
    g_feats = pl.pallas_call(
        _tc_readout,
        in_specs=[_full((NP, D)), _full((NP, 1))] + ro_specs + ro_specs,
        out_specs=_full((G, D)),
        out_shape=jax.ShapeDtypeStruct((G, D), jnp.float32),
    )(nf_final, gid_p, *ro_args)
    return g_feats
